# Initial kernel scaffold; baseline (speedup 1.0000x reference)
import jax
import jax.numpy as jnp
from jax.experimental import pallas as pl


def kernel(x, edge_index, batch, W0, b0, W1l, b1l, W1r, W2l, b2l, W2r, We0, be0, We1, be1, We2, be2, Wm, bm, Wg, bg, Wv, bv, Wo, bo):
    raise NotImplementedError("write your pallas kernel here")


# speedup vs baseline: 8.3152x; 8.3152x over previous
import functools

import jax
import jax.numpy as jnp
from jax import lax
from jax.experimental import pallas as pl
from jax.experimental.pallas import tpu as pltpu
from jax.experimental.pallas import tpu_sc as plsc

N = 50000
E = 800000
NG = 256
DIN = 128
DH = 64
HF = 32
GRPH = 64

EP = 819200
EROWS = EP // 128
NC = 2
NS = 16
RPT = EROWS // (NS)
K = 4
CHUNKS = RPT // K
ACC_ROWS = 50176
ZROWS = 56
APT = ACC_ROWS // NS

RB = 2000
GB = N // RB


def _mm0_body(x_ref, w_ref, b_ref, o_ref):
    y = jnp.dot(x_ref[...], w_ref[...], preferred_element_type=jnp.float32)
    y = y + b_ref[...]
    o_ref[0] = y[:, :HF]
    o_ref[1] = y[:, HF:]


def _dense0(x, W0, b0):
    return pl.pallas_call(
        _mm0_body,
        grid=(GB,),
        in_specs=[
            pl.BlockSpec((RB, DIN), lambda i: (i, 0)),
            pl.BlockSpec((DIN, DH), lambda i: (0, 0)),
            pl.BlockSpec((1, DH), lambda i: (0, 0)),
        ],
        out_specs=pl.BlockSpec((2, RB, HF), lambda i: (0, i, 0)),
        out_shape=jax.ShapeDtypeStruct((2, N, HF), jnp.float32),
    )(x, W0, b0.reshape(1, DH))


def _sage_body(aa_ref, ab_ref, deg_ref, xa_ref, xb_ref, wlt_ref, wlb_ref,
               b_ref, wrt_ref, wrb_ref, o_ref):
    inv = 1.0 / jnp.maximum(deg_ref[...], 1.0)
    y = jnp.dot(aa_ref[0] * inv, wlt_ref[...], preferred_element_type=jnp.float32)
    y = y + jnp.dot(ab_ref[0] * inv, wlb_ref[...], preferred_element_type=jnp.float32)
    y = y + jnp.dot(xa_ref[0], wrt_ref[...], preferred_element_type=jnp.float32)
    y = y + jnp.dot(xb_ref[0], wrb_ref[...], preferred_element_type=jnp.float32)
    y = y + b_ref[...]
    o_ref[0] = y[:, :HF]
    o_ref[1] = y[:, HF:]


def _dense_sage(aggs, deg2d, xs, Wl, bl, Wr):
    half = pl.BlockSpec((1, RB, HF), lambda i: (0, i, 0))
    half2 = pl.BlockSpec((1, RB, HF), lambda i: (1, i, 0))
    wspec = pl.BlockSpec((HF, DH), lambda i: (0, 0))
    return pl.pallas_call(
        _sage_body,
        grid=(GB,),
        in_specs=[
            half, half2,
            pl.BlockSpec((RB, 1), lambda i: (i, 0)),
            half, half2,
            wspec, wspec,
            pl.BlockSpec((1, DH), lambda i: (0, 0)),
            wspec, wspec,
        ],
        out_specs=pl.BlockSpec((2, RB, HF), lambda i: (0, i, 0)),
        out_shape=jax.ShapeDtypeStruct((2, N, HF), jnp.float32),
    )(aggs, aggs, deg2d, xs, xs, Wl[:HF], Wl[HF:], bl.reshape(1, DH),
      Wr[:HF], Wr[HF:])


def _make_sc_agg(with_deg):
    mesh = plsc.VectorSubcoreMesh(core_axis_name="c", subcore_axis_name="s")
    out_type = [jax.ShapeDtypeStruct((2, ACC_ROWS, HF), jnp.float32)]
    scratch = [
        pltpu.VMEM((K, 128), jnp.int32),
        pltpu.VMEM((K, 128), jnp.int32),
        pltpu.VMEM((K, 128, HF), jnp.float32),
        pltpu.VMEM((ZROWS, HF), jnp.float32),
        pltpu.VMEM_SHARED((ACC_ROWS, HF), jnp.float32),
        pltpu.SemaphoreType.DMA,
    ]
    if with_deg:
        out_type.append(jax.ShapeDtypeStruct((ACC_ROWS,), jnp.float32))
        scratch += [
            pltpu.VMEM((784,), jnp.float32),
            pltpu.VMEM((128,), jnp.float32),
            pltpu.VMEM_SHARED((ACC_ROWS,), jnp.float32),
        ]

    def body(tbl_ref, src_ref, dst_ref, out_ref, *rest):
        if with_deg:
            (deg_out, sbuf, dbuf, rows, zbuf, acc, sem, zbuf1, ones, dacc) = rest
        else:
            (sbuf, dbuf, rows, zbuf, acc, sem) = rest
        c = lax.axis_index("c")
        s = lax.axis_index("s")
        mytbl = tbl_ref.at[c]
        myout = out_ref.at[c]

        zero16 = jnp.zeros((16,), jnp.float32)

        @pl.loop(0, ZROWS)
        def _zfill(r):
            zbuf[r, pl.ds(0, 16)] = zero16
            zbuf[r, pl.ds(16, 16)] = zero16

        @pl.loop(0, 56)
        def _zacc(r):
            pltpu.sync_copy(zbuf, acc.at[pl.ds(s * 3136 + r * ZROWS, ZROWS)])

        if with_deg:
            @pl.when(c == 0)
            def _zdeg():
                @pl.loop(0, 49)
                def _zf1(r):
                    zbuf1[pl.ds(r * 16, 16)] = zero16
                one16 = jnp.ones((16,), jnp.float32)

                @pl.loop(0, 8)
                def _of(r):
                    ones[pl.ds(r * 16, 16)] = one16
                @pl.loop(0, 4)
                def _zd(r):
                    pltpu.sync_copy(zbuf1, dacc.at[pl.ds(s * 3136 + r * 784, 784)])

        plsc.subcore_barrier()

        @pl.loop(0, CHUNKS)
        def _chunk(g):
            base = s * RPT + g * K
            pltpu.sync_copy(src_ref.at[pl.ds(base, K)], sbuf)
            pltpu.sync_copy(dst_ref.at[pl.ds(base, K)], dbuf)
            descs = [
                pltpu.async_copy(mytbl.at[sbuf.at[j]], rows.at[j], sem)
                for j in range(K)
            ]
            for d in descs:
                d.wait()
            for j in range(K):
                pltpu.sync_copy(rows.at[j], acc.at[dbuf.at[j]], add=True)
            if with_deg:
                @pl.when(c == 0)
                def _deg():
                    for j in range(K):
                        pltpu.sync_copy(ones, dacc.at[dbuf.at[j]], add=True)

        plsc.subcore_barrier()

        pltpu.sync_copy(acc.at[pl.ds(s * APT, APT)], myout.at[pl.ds(s * APT, APT)])
        if with_deg:
            @pl.when(c == 0)
            def _wdeg():
                pltpu.sync_copy(dacc.at[pl.ds(s * 3136, 3136)],
                                deg_out.at[pl.ds(s * 3136, 3136)])

    return pl.kernel(body, out_type=out_type, mesh=mesh, scratch_types=scratch,
                     compiler_params=pltpu.CompilerParams(
                         use_tc_tiling_on_sc=False))


_make_sc_agg = functools.cache(_make_sc_agg)


def _final_body(x0a, x0b, x1a, x1b, x2a, x2b, bat_ref,
                we0t, we0b, be0, we1t, we1b, be1, we2t, we2b, be2,
                wm0, wm1, wm2, bm, wg, bg, wv, bv, wo, bo,
                out_ref, m_ref, gsum_ref, pooled_ref):
    i = pl.program_id(0)

    @pl.when(i == 0)
    def _init():
        m_ref[...] = jnp.full((NG, 1), -1e30, jnp.float32)
        gsum_ref[...] = jnp.zeros((NG, 1), jnp.float32)
        pooled_ref[...] = jnp.zeros((NG, GRPH), jnp.float32)

    def mm(a, b):
        return jnp.dot(a, b, preferred_element_type=jnp.float32)

    e0 = jax.nn.relu(mm(x0a[0], we0t[...]) + mm(x0b[0], we0b[...]) + be0[...])
    e1 = jax.nn.relu(mm(x1a[0], we1t[...]) + mm(x1b[0], we1b[...]) + be1[...])
    e2 = jax.nn.relu(mm(x2a[0], we2t[...]) + mm(x2b[0], we2b[...]) + be2[...])
    xc = jax.nn.relu(mm(e0, wm0[...]) + mm(e1, wm1[...]) + mm(e2, wm2[...])
                     + bm[...])
    gate = lax.dot_general(wg[...], xc, (((0,), (1,)), ((), ())),
                           preferred_element_type=jnp.float32) + bg[...]
    v = mm(xc, wv[...]) + bv[...]

    b_blk = bat_ref[0]
    gids = lax.broadcasted_iota(jnp.int32, (NG, RB), 0)
    onehot = (gids == b_blk).astype(jnp.float32)
    scores = jnp.where(gids == b_blk, gate, -1e30)
    bmax = jnp.max(scores, axis=1, keepdims=True)
    m_old = m_ref[...]
    m_new = jnp.maximum(m_old, bmax)
    scale = jnp.exp(m_old - m_new)
    m_node = lax.dot_general(m_new, onehot, (((0,), (0,)), ((), ())),
                             preferred_element_type=jnp.float32)
    w = jnp.exp(gate - m_node)
    wmat = onehot * w
    m_ref[...] = m_new
    gsum_ref[...] = gsum_ref[...] * scale + jnp.sum(wmat, axis=1, keepdims=True)
    pooled_ref[...] = pooled_ref[...] * scale + mm(wmat, v)

    @pl.when(i == GB - 1)
    def _fin():
        pooled = pooled_ref[...] / (gsum_ref[...] + 1e-16)
        logits = mm(pooled, wo[...]) + bo[...]
        lane = lax.broadcasted_iota(jnp.int32, (NG, 128), 1)
        logits = jnp.where(lane < 2, logits, -1e30)
        mx = jnp.max(logits, axis=1, keepdims=True)
        p = jnp.exp(logits - mx)
        out_ref[...] = p / jnp.sum(p, axis=1, keepdims=True)


def _final(x0s, x1s, x2s, batch, We0, be0, We1, be1, We2, be2, Wm, bm,
           Wg, bg, Wv, bv, Wo, bo):
    half = pl.BlockSpec((1, RB, HF), lambda i: (0, i, 0))
    half2 = pl.BlockSpec((1, RB, HF), lambda i: (1, i, 0))
    wspec = pl.BlockSpec((HF, GRPH), lambda i: (0, 0))
    bspec = pl.BlockSpec((1, GRPH), lambda i: (0, 0))
    mspec = pl.BlockSpec((DH, GRPH), lambda i: (0, 0))
    Wo128 = jnp.pad(Wo, ((0, 0), (0, 128 - Wo.shape[1])))
    bo128 = jnp.pad(bo.reshape(1, -1), ((0, 0), (0, 128 - bo.shape[0])))
    out = pl.pallas_call(
        _final_body,
        grid=(GB,),
        in_specs=[
            half, half2, half, half2, half, half2,
            pl.BlockSpec((1, 1, RB), lambda i: (i, 0, 0)),
            wspec, wspec, bspec, wspec, wspec, bspec, wspec, wspec, bspec,
            mspec, mspec, mspec, bspec,
            pl.BlockSpec((DH, 1), lambda i: (0, 0)),
            pl.BlockSpec((1, 1), lambda i: (0, 0)),
            pl.BlockSpec((DH, GRPH), lambda i: (0, 0)),
            bspec,
            pl.BlockSpec((DH, 128), lambda i: (0, 0)),
            pl.BlockSpec((1, 128), lambda i: (0, 0)),
        ],
        out_specs=pl.BlockSpec((NG, 128), lambda i: (0, 0)),
        out_shape=jax.ShapeDtypeStruct((NG, 128), jnp.float32),
        compiler_params=pltpu.CompilerParams(
            dimension_semantics=("arbitrary",)),
        scratch_shapes=[
            pltpu.VMEM((NG, 1), jnp.float32),
            pltpu.VMEM((NG, 1), jnp.float32),
            pltpu.VMEM((NG, GRPH), jnp.float32),
        ],
    )(x0s, x0s, x1s, x1s, x2s, x2s, batch.reshape(GB, 1, RB),
      We0[:HF], We0[HF:], be0.reshape(1, GRPH),
      We1[:HF], We1[HF:], be1.reshape(1, GRPH),
      We2[:HF], We2[HF:], be2.reshape(1, GRPH),
      Wm[:DH], Wm[DH:2 * DH], Wm[2 * DH:], bm.reshape(1, GRPH),
      Wg, bg.reshape(1, 1), Wv, bv.reshape(1, GRPH), Wo128, bo128)
    return out[:, :2]


def _sage_agg(xs, src2d, dst2d, with_deg):
    if with_deg:
        return tuple(_make_sc_agg(True)(xs, src2d, dst2d))
    return tuple(_make_sc_agg(False)(xs, src2d, dst2d))


def kernel(x, edge_index, batch, W0, b0, W1l, b1l, W1r, W2l, b2l, W2r,
           We0, be0, We1, be1, We2, be2, Wm, bm, Wg, bg, Wv, bv, Wo, bo):
    src = edge_index[0]
    dst = edge_index[1]
    pad_i = jnp.arange(EP - E, dtype=jnp.int32)
    src2d = jnp.concatenate([src, pad_i % N]).reshape(EROWS, 128)
    dst2d = jnp.concatenate([dst, N + pad_i % (ACC_ROWS - N)]).reshape(EROWS, 128)

    x0s = _dense0(x, W0, b0)
    agg1, degp = _sage_agg(x0s, src2d, dst2d, True)
    deg2d = degp[:N].reshape(N, 1)
    x1s = _dense_sage(agg1, deg2d, x0s, W1l, b1l, W1r)
    (agg2,) = _sage_agg(x1s, src2d, dst2d, False)
    x2s = _dense_sage(agg2, deg2d, x1s, W2l, b2l, W2r)
    return _final(x0s, x1s, x2s, batch, We0, be0, We1, be1, We2, be2,
                  Wm, bm, Wg, bg, Wv, bv, Wo, bo)


# speedup vs baseline: 10.5856x; 1.2731x over previous
import functools

import jax
import jax.numpy as jnp
from jax import lax
from jax.experimental import pallas as pl
from jax.experimental.pallas import tpu as pltpu
from jax.experimental.pallas import tpu_sc as plsc

N = 50000
E = 800000
NG = 256
DIN = 128
DH = 64
HF = 32
GRPH = 64

EP = 819200
EROWS = EP // 128
NC = 2
NS = 16
RPT = EROWS // (NS)
K = 2
CHUNKS = RPT // K
HPAIR = CHUNKS // 2
ACC_ROWS = 50176
ZROWS = 56
APT = ACC_ROWS // NS

RB = 2000
GB = N // RB


def _mm0_body(x_ref, w_ref, b_ref, o_ref):
    y = jnp.dot(x_ref[...], w_ref[...], preferred_element_type=jnp.float32)
    y = y + b_ref[...]
    o_ref[0] = y[:, :HF]
    o_ref[1] = y[:, HF:]


def _dense0(x, W0, b0):
    return pl.pallas_call(
        _mm0_body,
        grid=(GB,),
        in_specs=[
            pl.BlockSpec((RB, DIN), lambda i: (i, 0)),
            pl.BlockSpec((DIN, DH), lambda i: (0, 0)),
            pl.BlockSpec((1, DH), lambda i: (0, 0)),
        ],
        out_specs=pl.BlockSpec((2, RB, HF), lambda i: (0, i, 0)),
        out_shape=jax.ShapeDtypeStruct((2, N, HF), jnp.float32),
    )(x, W0, b0.reshape(1, DH))


def _sage_body(aa_ref, ab_ref, deg_ref, xa_ref, xb_ref, wlt_ref, wlb_ref,
               b_ref, wrt_ref, wrb_ref, o_ref):
    inv = 1.0 / jnp.maximum(deg_ref[...], 1.0)
    y = jnp.dot(aa_ref[0] * inv, wlt_ref[...], preferred_element_type=jnp.float32)
    y = y + jnp.dot(ab_ref[0] * inv, wlb_ref[...], preferred_element_type=jnp.float32)
    y = y + jnp.dot(xa_ref[0], wrt_ref[...], preferred_element_type=jnp.float32)
    y = y + jnp.dot(xb_ref[0], wrb_ref[...], preferred_element_type=jnp.float32)
    y = y + b_ref[...]
    o_ref[0] = y[:, :HF]
    o_ref[1] = y[:, HF:]


def _dense_sage(aggs, deg2d, xs, Wl, bl, Wr):
    half = pl.BlockSpec((1, RB, HF), lambda i: (0, i, 0))
    half2 = pl.BlockSpec((1, RB, HF), lambda i: (1, i, 0))
    wspec = pl.BlockSpec((HF, DH), lambda i: (0, 0))
    return pl.pallas_call(
        _sage_body,
        grid=(GB,),
        in_specs=[
            half, half2,
            pl.BlockSpec((RB, 1), lambda i: (i, 0)),
            half, half2,
            wspec, wspec,
            pl.BlockSpec((1, DH), lambda i: (0, 0)),
            wspec, wspec,
        ],
        out_specs=pl.BlockSpec((2, RB, HF), lambda i: (0, i, 0)),
        out_shape=jax.ShapeDtypeStruct((2, N, HF), jnp.float32),
    )(aggs, aggs, deg2d, xs, xs, Wl[:HF], Wl[HF:], bl.reshape(1, DH),
      Wr[:HF], Wr[HF:])


def _make_sc_agg(with_deg):
    mesh = plsc.VectorSubcoreMesh(core_axis_name="c", subcore_axis_name="s")
    out_type = [jax.ShapeDtypeStruct((2, ACC_ROWS, HF), jnp.float32)]
    scratch = [
        pltpu.VMEM((K, 2, 128), jnp.int32),
        pltpu.VMEM((K, 2, 128), jnp.int32),
        pltpu.VMEM((K, 128, HF), jnp.float32),
        pltpu.VMEM((K, 128, HF), jnp.float32),
        pltpu.VMEM((ZROWS, HF), jnp.float32),
        pltpu.VMEM_SHARED((ACC_ROWS, HF), jnp.float32),
        pltpu.SemaphoreType.DMA,
        pltpu.SemaphoreType.DMA,
        pltpu.SemaphoreType.DMA,
        pltpu.SemaphoreType.DMA,
        pltpu.SemaphoreType.DMA,
        pltpu.SemaphoreType.DMA,
    ]
    if with_deg:
        out_type.append(jax.ShapeDtypeStruct((ACC_ROWS,), jnp.float32))
        scratch += [
            pltpu.VMEM((784,), jnp.float32),
            pltpu.VMEM((128,), jnp.float32),
            pltpu.VMEM_SHARED((ACC_ROWS,), jnp.float32),
        ]

    def body(tbl_ref, ei_ref, out_ref, *rest):
        if with_deg:
            (deg_out, sdA, sdB, rowsA, rowsB, zbuf, acc,
             isemA, isemB, gsemA, gsemB, ssemA, ssemB,
             zbuf1, ones, dacc) = rest
        else:
            (sdA, sdB, rowsA, rowsB, zbuf, acc,
             isemA, isemB, gsemA, gsemB, ssemA, ssemB) = rest
        c = lax.axis_index("c")
        s = lax.axis_index("s")
        mytbl = tbl_ref.at[c]
        myout = out_ref.at[c]

        zero16 = jnp.zeros((16,), jnp.float32)

        @pl.loop(0, ZROWS)
        def _zfill(r):
            zbuf[r, pl.ds(0, 16)] = zero16
            zbuf[r, pl.ds(16, 16)] = zero16

        @pl.loop(0, 56)
        def _zacc(r):
            pltpu.sync_copy(zbuf, acc.at[pl.ds(s * 3136 + r * ZROWS, ZROWS)])

        if with_deg:
            @pl.when(c == 0)
            def _zdeg():
                @pl.loop(0, 49)
                def _zf1(r):
                    zbuf1[pl.ds(r * 16, 16)] = zero16
                one16 = jnp.ones((16,), jnp.float32)

                @pl.loop(0, 8)
                def _of(r):
                    ones[pl.ds(r * 16, 16)] = one16
                @pl.loop(0, 4)
                def _zd(r):
                    pltpu.sync_copy(zbuf1, dacc.at[pl.ds(s * 3136 + r * 784, 784)])

        plsc.subcore_barrier()

        def fire_idx(chunk_base, sd, isem):
            return pltpu.async_copy(ei_ref.at[pl.ds(chunk_base, K)], sd, isem)

        def fire_gathers(sd, rows, gsem):
            for j in range(K):
                pltpu.async_copy(mytbl.at[sd.at[j, 0]], rows.at[j], gsem)

        def wait_gathers(sd, rows, gsem):
            for j in range(K):
                pltpu.make_async_copy(mytbl.at[sd.at[j, 0]], rows.at[j],
                                      gsem).wait()

        def fire_scatters(sd, rows, ssem):
            for j in range(K):
                pltpu.async_copy(rows.at[j], acc.at[sd.at[j, 1]], ssem,
                                 add=True)
            if with_deg:
                @pl.when(c == 0)
                def _dfire():
                    for j in range(K):
                        pltpu.async_copy(ones, dacc.at[sd.at[j, 1]], ssem,
                                         add=True)

        def wait_scatters(sd, rows, ssem):
            for j in range(K):
                pltpu.make_async_copy(rows.at[j], acc.at[sd.at[j, 1]],
                                      ssem).wait()
            if with_deg:
                @pl.when(c == 0)
                def _dwait():
                    for j in range(K):
                        pltpu.make_async_copy(ones, dacc.at[sd.at[j, 1]],
                                              ssem).wait()

        tbase = s * RPT
        fire_idx(tbase, sdA, isemA).wait()
        fire_gathers(sdA, rowsA, gsemA)
        fire_idx(tbase + K, sdB, isemB).wait()
        fire_gathers(sdB, rowsB, gsemB)

        @pl.loop(0, HPAIR)
        def _pair(h):
            be = tbase + (2 * h) * K
            wait_gathers(sdA, rowsA, gsemA)
            fire_scatters(sdA, rowsA, ssemA)
            wait_gathers(sdB, rowsB, gsemB)
            fire_scatters(sdB, rowsB, ssemB)
            wait_scatters(sdA, rowsA, ssemA)

            @pl.when(h < HPAIR - 1)
            def _nextA():
                fire_idx(be + 2 * K, sdA, isemA).wait()
                fire_gathers(sdA, rowsA, gsemA)
            wait_scatters(sdB, rowsB, ssemB)

            @pl.when(h < HPAIR - 1)
            def _nextB():
                fire_idx(be + 3 * K, sdB, isemB).wait()
                fire_gathers(sdB, rowsB, gsemB)

        plsc.subcore_barrier()

        pltpu.sync_copy(acc.at[pl.ds(s * APT, APT)], myout.at[pl.ds(s * APT, APT)])
        if with_deg:
            @pl.when(c == 0)
            def _wdeg():
                pltpu.sync_copy(dacc.at[pl.ds(s * 3136, 3136)],
                                deg_out.at[pl.ds(s * 3136, 3136)])

    return pl.kernel(body, out_type=out_type, mesh=mesh, scratch_types=scratch,
                     compiler_params=pltpu.CompilerParams(
                         use_tc_tiling_on_sc=False))


_make_sc_agg = functools.cache(_make_sc_agg)


def _final_body(x0a, x0b, x1a, x1b, x2a, x2b, bat_ref,
                we0t, we0b, be0, we1t, we1b, be1, we2t, we2b, be2,
                wm0, wm1, wm2, bm, wg, bg, wv, bv, wo, bo,
                out_ref, m_ref, gsum_ref, pooled_ref):
    i = pl.program_id(0)

    @pl.when(i == 0)
    def _init():
        m_ref[...] = jnp.full((NG, 1), -1e30, jnp.float32)
        gsum_ref[...] = jnp.zeros((NG, 1), jnp.float32)
        pooled_ref[...] = jnp.zeros((NG, GRPH), jnp.float32)

    def mm(a, b):
        return jnp.dot(a, b, preferred_element_type=jnp.float32)

    e0 = jax.nn.relu(mm(x0a[0], we0t[...]) + mm(x0b[0], we0b[...]) + be0[...])
    e1 = jax.nn.relu(mm(x1a[0], we1t[...]) + mm(x1b[0], we1b[...]) + be1[...])
    e2 = jax.nn.relu(mm(x2a[0], we2t[...]) + mm(x2b[0], we2b[...]) + be2[...])
    xc = jax.nn.relu(mm(e0, wm0[...]) + mm(e1, wm1[...]) + mm(e2, wm2[...])
                     + bm[...])
    gate = lax.dot_general(wg[...], xc, (((0,), (1,)), ((), ())),
                           preferred_element_type=jnp.float32) + bg[...]
    v = mm(xc, wv[...]) + bv[...]

    b_blk = bat_ref[0]
    gids = lax.broadcasted_iota(jnp.int32, (NG, RB), 0)
    onehot = (gids == b_blk).astype(jnp.float32)
    scores = jnp.where(gids == b_blk, gate, -1e30)
    bmax = jnp.max(scores, axis=1, keepdims=True)
    m_old = m_ref[...]
    m_new = jnp.maximum(m_old, bmax)
    scale = jnp.exp(m_old - m_new)
    m_node = lax.dot_general(m_new, onehot, (((0,), (0,)), ((), ())),
                             preferred_element_type=jnp.float32)
    w = jnp.exp(gate - m_node)
    wmat = onehot * w
    m_ref[...] = m_new
    gsum_ref[...] = gsum_ref[...] * scale + jnp.sum(wmat, axis=1, keepdims=True)
    pooled_ref[...] = pooled_ref[...] * scale + mm(wmat, v)

    @pl.when(i == GB - 1)
    def _fin():
        pooled = pooled_ref[...] / (gsum_ref[...] + 1e-16)
        logits = mm(pooled, wo[...]) + bo[...]
        lane = lax.broadcasted_iota(jnp.int32, (NG, 128), 1)
        logits = jnp.where(lane < 2, logits, -1e30)
        mx = jnp.max(logits, axis=1, keepdims=True)
        p = jnp.exp(logits - mx)
        out_ref[...] = p / jnp.sum(p, axis=1, keepdims=True)


def _final(x0s, x1s, x2s, batch, We0, be0, We1, be1, We2, be2, Wm, bm,
           Wg, bg, Wv, bv, Wo, bo):
    half = pl.BlockSpec((1, RB, HF), lambda i: (0, i, 0))
    half2 = pl.BlockSpec((1, RB, HF), lambda i: (1, i, 0))
    wspec = pl.BlockSpec((HF, GRPH), lambda i: (0, 0))
    bspec = pl.BlockSpec((1, GRPH), lambda i: (0, 0))
    mspec = pl.BlockSpec((DH, GRPH), lambda i: (0, 0))
    Wo128 = jnp.pad(Wo, ((0, 0), (0, 128 - Wo.shape[1])))
    bo128 = jnp.pad(bo.reshape(1, -1), ((0, 0), (0, 128 - bo.shape[0])))
    out = pl.pallas_call(
        _final_body,
        grid=(GB,),
        in_specs=[
            half, half2, half, half2, half, half2,
            pl.BlockSpec((1, 1, RB), lambda i: (i, 0, 0)),
            wspec, wspec, bspec, wspec, wspec, bspec, wspec, wspec, bspec,
            mspec, mspec, mspec, bspec,
            pl.BlockSpec((DH, 1), lambda i: (0, 0)),
            pl.BlockSpec((1, 1), lambda i: (0, 0)),
            pl.BlockSpec((DH, GRPH), lambda i: (0, 0)),
            bspec,
            pl.BlockSpec((DH, 128), lambda i: (0, 0)),
            pl.BlockSpec((1, 128), lambda i: (0, 0)),
        ],
        out_specs=pl.BlockSpec((NG, 128), lambda i: (0, 0)),
        out_shape=jax.ShapeDtypeStruct((NG, 128), jnp.float32),
        compiler_params=pltpu.CompilerParams(
            dimension_semantics=("arbitrary",)),
        scratch_shapes=[
            pltpu.VMEM((NG, 1), jnp.float32),
            pltpu.VMEM((NG, 1), jnp.float32),
            pltpu.VMEM((NG, GRPH), jnp.float32),
        ],
    )(x0s, x0s, x1s, x1s, x2s, x2s, batch.reshape(GB, 1, RB),
      We0[:HF], We0[HF:], be0.reshape(1, GRPH),
      We1[:HF], We1[HF:], be1.reshape(1, GRPH),
      We2[:HF], We2[HF:], be2.reshape(1, GRPH),
      Wm[:DH], Wm[DH:2 * DH], Wm[2 * DH:], bm.reshape(1, GRPH),
      Wg, bg.reshape(1, 1), Wv, bv.reshape(1, GRPH), Wo128, bo128)
    return out[:, :2]


def _sage_agg(xs, ei, with_deg):
    if with_deg:
        return tuple(_make_sc_agg(True)(xs, ei))
    return tuple(_make_sc_agg(False)(xs, ei))


def kernel(x, edge_index, batch, W0, b0, W1l, b1l, W1r, W2l, b2l, W2r,
           We0, be0, We1, be1, We2, be2, Wm, bm, Wg, bg, Wv, bv, Wo, bo):
    src = edge_index[0]
    dst = edge_index[1]
    pad_i = jnp.arange(EP - E, dtype=jnp.int32)
    src2d = jnp.concatenate([src, pad_i % N]).reshape(EROWS, 128)
    dst2d = jnp.concatenate([dst, N + pad_i % (ACC_ROWS - N)]).reshape(EROWS, 128)
    ei = jnp.stack([src2d, dst2d], axis=1)

    x0s = _dense0(x, W0, b0)
    agg1, degp = _sage_agg(x0s, ei, True)
    deg2d = degp[:N].reshape(N, 1)
    x1s = _dense_sage(agg1, deg2d, x0s, W1l, b1l, W1r)
    (agg2,) = _sage_agg(x1s, ei, False)
    x2s = _dense_sage(agg2, deg2d, x1s, W2l, b2l, W2r)
    return _final(x0s, x1s, x2s, batch, We0, be0, We1, be1, We2, be2,
                  Wm, bm, Wg, bg, Wv, bv, Wo, bo)


# speedup vs baseline: 12.9396x; 1.2224x over previous
import functools

import jax
import jax.numpy as jnp
from jax import lax
from jax.experimental import pallas as pl
from jax.experimental.pallas import tpu as pltpu
from jax.experimental.pallas import tpu_sc as plsc

N = 50000
E = 800000
NG = 256
DIN = 128
DH = 64
HF = 32
GRPH = 64

EP = 819200
EROWS = EP // 128
NC = 2
NS = 16
RPT = EROWS // (NS)
K = 2
CHUNKS = RPT // K
HPAIR = CHUNKS // 2
ACC_ROWS = 51200
ZROWS = 64
APT = ACC_ROWS // NS

RB = 2000
GB = N // RB
PB = 512
PK = ACC_ROWS // 4
PN = N // 4


def _bd4(W):
    return jnp.kron(jnp.eye(4, dtype=W.dtype), W)


def _t4(b):
    return jnp.tile(b, 4).reshape(1, -1)


def _mm0_body(x_ref, wa_ref, wb_ref, ba_ref, bb_ref, o_ref):
    xb = x_ref[...]
    o_ref[0] = jnp.dot(xb, wa_ref[...],
                       preferred_element_type=jnp.float32) + ba_ref[...]
    o_ref[1] = jnp.dot(xb, wb_ref[...],
                       preferred_element_type=jnp.float32) + bb_ref[...]


def _dense0(x, W0, b0):
    xp = jnp.pad(x.reshape(PN, 4 * DIN), ((0, PK - PN), (0, 0)))
    wspec = pl.BlockSpec((4 * DIN, 128), lambda i: (0, 0))
    bspec = pl.BlockSpec((1, 128), lambda i: (0, 0))
    return pl.pallas_call(
        _mm0_body,
        grid=(GB,),
        in_specs=[
            pl.BlockSpec((PB, 4 * DIN), lambda i: (i, 0)),
            wspec, wspec, bspec, bspec,
        ],
        out_specs=pl.BlockSpec((2, PB, 128), lambda i: (0, i, 0)),
        out_shape=jax.ShapeDtypeStruct((2, PK, 128), jnp.float32),
    )(xp, _bd4(W0[:, :HF]), _bd4(W0[:, HF:]), _t4(b0[:HF]), _t4(b0[HF:]))


def _sage_body(aa_ref, ab_ref, dinv_ref, xa_ref, xb_ref,
               wlaa, wlba, wraa, wrba, wlab, wlbb, wrab, wrbb,
               ba_ref, bb_ref, o_ref):
    def mm(a, b):
        return jnp.dot(a, b[...], preferred_element_type=jnp.float32)
    na = aa_ref[0] * dinv_ref[...]
    nb = ab_ref[0] * dinv_ref[...]
    xa = xa_ref[0]
    xb = xb_ref[0]
    o_ref[0] = (mm(na, wlaa) + mm(nb, wlba) + mm(xa, wraa) + mm(xb, wrba)
                + ba_ref[...])
    o_ref[1] = (mm(na, wlab) + mm(nb, wlbb) + mm(xa, wrab) + mm(xb, wrbb)
                + bb_ref[...])


def _dense_sage(aggs, dinvp, xs, Wl, bl, Wr):
    half = pl.BlockSpec((1, PB, 128), lambda i: (0, i, 0))
    half2 = pl.BlockSpec((1, PB, 128), lambda i: (1, i, 0))
    wspec = pl.BlockSpec((128, 128), lambda i: (0, 0))
    bspec = pl.BlockSpec((1, 128), lambda i: (0, 0))
    aggp = aggs.reshape(2, PK, 128)
    ws = [_bd4(Wl[:HF, :HF]), _bd4(Wl[HF:, :HF]),
          _bd4(Wr[:HF, :HF]), _bd4(Wr[HF:, :HF]),
          _bd4(Wl[:HF, HF:]), _bd4(Wl[HF:, HF:]),
          _bd4(Wr[:HF, HF:]), _bd4(Wr[HF:, HF:])]
    return pl.pallas_call(
        _sage_body,
        grid=(GB,),
        in_specs=[half, half2, pl.BlockSpec((PB, 128), lambda i: (i, 0)),
                  half, half2] + [wspec] * 8 + [bspec, bspec],
        out_specs=pl.BlockSpec((2, PB, 128), lambda i: (0, i, 0)),
        out_shape=jax.ShapeDtypeStruct((2, PK, 128), jnp.float32),
    )(aggp, aggp, dinvp, xs, xs, *ws, _t4(bl[:HF]), _t4(bl[HF:]))


def _make_sc_agg(with_deg):
    mesh = plsc.VectorSubcoreMesh(core_axis_name="c", subcore_axis_name="s")
    out_type = [jax.ShapeDtypeStruct((2, ACC_ROWS, HF), jnp.float32)]
    scratch = [
        pltpu.VMEM((K, 2, 128), jnp.int32),
        pltpu.VMEM((K, 2, 128), jnp.int32),
        pltpu.VMEM((K, 128, HF), jnp.float32),
        pltpu.VMEM((K, 128, HF), jnp.float32),
        pltpu.VMEM((ZROWS, HF), jnp.float32),
        pltpu.VMEM_SHARED((ACC_ROWS, HF), jnp.float32),
        pltpu.SemaphoreType.DMA,
        pltpu.SemaphoreType.DMA,
        pltpu.SemaphoreType.DMA,
        pltpu.SemaphoreType.DMA,
        pltpu.SemaphoreType.DMA,
        pltpu.SemaphoreType.DMA,
    ]
    if with_deg:
        out_type.append(jax.ShapeDtypeStruct((ACC_ROWS,), jnp.float32))
        scratch += [
            pltpu.VMEM((800,), jnp.float32),
            pltpu.VMEM((128,), jnp.float32),
            pltpu.VMEM_SHARED((ACC_ROWS,), jnp.float32),
        ]

    def body(tbl_ref, ei_ref, out_ref, *rest):
        if with_deg:
            (deg_out, sdA, sdB, rowsA, rowsB, zbuf, acc,
             isemA, isemB, gsemA, gsemB, ssemA, ssemB,
             zbuf1, ones, dacc) = rest
        else:
            (sdA, sdB, rowsA, rowsB, zbuf, acc,
             isemA, isemB, gsemA, gsemB, ssemA, ssemB) = rest
        c = lax.axis_index("c")
        s = lax.axis_index("s")
        mytbl = tbl_ref.at[c]
        myout = out_ref.at[c]

        zero16 = jnp.zeros((16,), jnp.float32)

        @pl.loop(0, ZROWS)
        def _zfill(r):
            zbuf[r, pl.ds(0, 16)] = zero16
            zbuf[r, pl.ds(16, 16)] = zero16

        @pl.loop(0, 50)
        def _zacc(r):
            pltpu.sync_copy(zbuf, acc.at[pl.ds(s * 3200 + r * ZROWS, ZROWS)])

        if with_deg:
            @pl.when(c == 0)
            def _zdeg():
                @pl.loop(0, 50)
                def _zf1(r):
                    zbuf1[pl.ds(r * 16, 16)] = zero16
                one16 = jnp.ones((16,), jnp.float32)

                @pl.loop(0, 8)
                def _of(r):
                    ones[pl.ds(r * 16, 16)] = one16
                @pl.loop(0, 4)
                def _zd(r):
                    pltpu.sync_copy(zbuf1, dacc.at[pl.ds(s * 3200 + r * 800, 800)])

        plsc.subcore_barrier()

        def fire_idx(chunk_base, sd, isem):
            return pltpu.async_copy(ei_ref.at[pl.ds(chunk_base, K)], sd, isem)

        def fire_gathers(sd, rows, gsem):
            for j in range(K):
                pltpu.async_copy(mytbl.at[sd.at[j, 0]], rows.at[j], gsem)

        def wait_gathers(sd, rows, gsem):
            for j in range(K):
                pltpu.make_async_copy(mytbl.at[sd.at[j, 0]], rows.at[j],
                                      gsem).wait()

        def fire_scatters(sd, rows, ssem):
            for j in range(K):
                pltpu.async_copy(rows.at[j], acc.at[sd.at[j, 1]], ssem,
                                 add=True)
            if with_deg:
                @pl.when(c == 0)
                def _dfire():
                    for j in range(K):
                        pltpu.async_copy(ones, dacc.at[sd.at[j, 1]], ssem,
                                         add=True)

        def wait_scatters(sd, rows, ssem):
            for j in range(K):
                pltpu.make_async_copy(rows.at[j], acc.at[sd.at[j, 1]],
                                      ssem).wait()
            if with_deg:
                @pl.when(c == 0)
                def _dwait():
                    for j in range(K):
                        pltpu.make_async_copy(ones, dacc.at[sd.at[j, 1]],
                                              ssem).wait()

        tbase = s * RPT
        fire_idx(tbase, sdA, isemA).wait()
        fire_gathers(sdA, rowsA, gsemA)
        fire_idx(tbase + K, sdB, isemB).wait()
        fire_gathers(sdB, rowsB, gsemB)

        @pl.loop(0, HPAIR)
        def _pair(h):
            be = tbase + (2 * h) * K
            wait_gathers(sdA, rowsA, gsemA)
            fire_scatters(sdA, rowsA, ssemA)
            wait_gathers(sdB, rowsB, gsemB)
            fire_scatters(sdB, rowsB, ssemB)
            wait_scatters(sdA, rowsA, ssemA)

            @pl.when(h < HPAIR - 1)
            def _nextA():
                fire_idx(be + 2 * K, sdA, isemA).wait()
                fire_gathers(sdA, rowsA, gsemA)
            wait_scatters(sdB, rowsB, ssemB)

            @pl.when(h < HPAIR - 1)
            def _nextB():
                fire_idx(be + 3 * K, sdB, isemB).wait()
                fire_gathers(sdB, rowsB, gsemB)

        plsc.subcore_barrier()

        pltpu.sync_copy(acc.at[pl.ds(s * APT, APT)], myout.at[pl.ds(s * APT, APT)])
        if with_deg:
            @pl.when(c == 0)
            def _wdeg():
                pltpu.sync_copy(dacc.at[pl.ds(s * 3200, 3200)],
                                deg_out.at[pl.ds(s * 3200, 3200)])

    return pl.kernel(body, out_type=out_type, mesh=mesh, scratch_types=scratch,
                     compiler_params=pltpu.CompilerParams(
                         use_tc_tiling_on_sc=False))


_make_sc_agg = functools.cache(_make_sc_agg)


def _final_body(x0a, x0b, x1a, x1b, x2a, x2b, bat_ref,
                we0t, we0b, be0, we1t, we1b, be1, we2t, we2b, be2,
                wm0, wm1, wm2, bm, wg, bg, wv, bv, wo, bo,
                out_ref, m_ref, gsum_ref, pooled_ref):
    i = pl.program_id(0)

    @pl.when(i == 0)
    def _init():
        m_ref[...] = jnp.full((NG, 1), -1e30, jnp.float32)
        gsum_ref[...] = jnp.zeros((NG, 1), jnp.float32)
        pooled_ref[...] = jnp.zeros((NG, GRPH), jnp.float32)

    def mm(a, b):
        return jnp.dot(a, b, preferred_element_type=jnp.float32)

    e0 = jax.nn.relu(mm(x0a[0], we0t[...]) + mm(x0b[0], we0b[...]) + be0[...])
    e1 = jax.nn.relu(mm(x1a[0], we1t[...]) + mm(x1b[0], we1b[...]) + be1[...])
    e2 = jax.nn.relu(mm(x2a[0], we2t[...]) + mm(x2b[0], we2b[...]) + be2[...])
    xc = jax.nn.relu(mm(e0, wm0[...]) + mm(e1, wm1[...]) + mm(e2, wm2[...])
                     + bm[...])
    v = mm(xc, wv[...]) + bv[...]

    gids = lax.broadcasted_iota(jnp.int32, (NG, PB), 0)
    m_old = m_ref[...]
    gates, onehots, bmaxs = [], [], []
    for k in range(4):
        xck = xc[:, k * GRPH:(k + 1) * GRPH]
        gate_k = lax.dot_general(wg[...], xck, (((0,), (1,)), ((), ())),
                                 preferred_element_type=jnp.float32) + bg[...]
        b_k = bat_ref[0, pl.ds(k, 1)]
        eq_k = gids == b_k
        gates.append(gate_k)
        onehots.append(eq_k.astype(jnp.float32))
        bmaxs.append(jnp.max(jnp.where(eq_k, gate_k, -1e30), axis=1,
                             keepdims=True))
    m_new = jnp.maximum(jnp.maximum(jnp.maximum(m_old, bmaxs[0]),
                                    jnp.maximum(bmaxs[1], bmaxs[2])),
                        bmaxs[3])
    scale = jnp.exp(m_old - m_new)
    gsum_inc = jnp.zeros((NG, 1), jnp.float32)
    pooled_inc = jnp.zeros((NG, GRPH), jnp.float32)
    for k in range(4):
        m_node = lax.dot_general(m_new, onehots[k], (((0,), (0,)), ((), ())),
                                 preferred_element_type=jnp.float32)
        wmat = jnp.where(onehots[k] > 0.0,
                         jnp.exp(gates[k] - m_node), 0.0)
        gsum_inc = gsum_inc + jnp.sum(wmat, axis=1, keepdims=True)
        pooled_inc = pooled_inc + mm(wmat, v[:, k * GRPH:(k + 1) * GRPH])
    m_ref[...] = m_new
    gsum_ref[...] = gsum_ref[...] * scale + gsum_inc
    pooled_ref[...] = pooled_ref[...] * scale + pooled_inc

    @pl.when(i == GB - 1)
    def _fin():
        pooled = pooled_ref[...] / (gsum_ref[...] + 1e-16)
        logits = mm(pooled, wo[...]) + bo[...]
        lane = lax.broadcasted_iota(jnp.int32, (NG, 128), 1)
        logits = jnp.where(lane < 2, logits, -1e30)
        mx = jnp.max(logits, axis=1, keepdims=True)
        p = jnp.exp(logits - mx)
        out_ref[...] = p / jnp.sum(p, axis=1, keepdims=True)


def _final(x0s, x1s, x2s, batch, We0, be0, We1, be1, We2, be2, Wm, bm,
           Wg, bg, Wv, bv, Wo, bo):
    half = pl.BlockSpec((1, PB, 128), lambda i: (0, i, 0))
    half2 = pl.BlockSpec((1, PB, 128), lambda i: (1, i, 0))
    wspec = pl.BlockSpec((128, 4 * GRPH), lambda i: (0, 0))
    bspec = pl.BlockSpec((1, 4 * GRPH), lambda i: (0, 0))
    mspec = pl.BlockSpec((4 * DH, 4 * GRPH), lambda i: (0, 0))
    Wo128 = jnp.pad(Wo, ((0, 0), (0, 128 - Wo.shape[1])))
    bo128 = jnp.pad(bo.reshape(1, -1), ((0, 0), (0, 128 - bo.shape[0])))
    batp = jnp.concatenate(
        [batch, jnp.full((4 * PK - N,), -1, jnp.int32)]
    ).reshape(GB, PB, 4).transpose(0, 2, 1)
    out = pl.pallas_call(
        _final_body,
        grid=(GB,),
        in_specs=[
            half, half2, half, half2, half, half2,
            pl.BlockSpec((1, 4, PB), lambda i: (i, 0, 0)),
            wspec, wspec, bspec, wspec, wspec, bspec, wspec, wspec, bspec,
            mspec, mspec, mspec, bspec,
            pl.BlockSpec((DH, 1), lambda i: (0, 0)),
            pl.BlockSpec((1, 1), lambda i: (0, 0)),
            mspec,
            bspec,
            pl.BlockSpec((DH, 128), lambda i: (0, 0)),
            pl.BlockSpec((1, 128), lambda i: (0, 0)),
        ],
        out_specs=pl.BlockSpec((NG, 128), lambda i: (0, 0)),
        out_shape=jax.ShapeDtypeStruct((NG, 128), jnp.float32),
        compiler_params=pltpu.CompilerParams(
            dimension_semantics=("arbitrary",)),
        scratch_shapes=[
            pltpu.VMEM((NG, 1), jnp.float32),
            pltpu.VMEM((NG, 1), jnp.float32),
            pltpu.VMEM((NG, GRPH), jnp.float32),
        ],
    )(x0s, x0s, x1s, x1s, x2s, x2s, batp,
      _bd4(We0[:HF]), _bd4(We0[HF:]), _t4(be0),
      _bd4(We1[:HF]), _bd4(We1[HF:]), _t4(be1),
      _bd4(We2[:HF]), _bd4(We2[HF:]), _t4(be2),
      _bd4(Wm[:DH]), _bd4(Wm[DH:2 * DH]), _bd4(Wm[2 * DH:]), _t4(bm),
      Wg, bg.reshape(1, 1), _bd4(Wv), _t4(bv), Wo128, bo128)
    return out[:, :2]


def _sage_agg(xs, ei, with_deg):
    if with_deg:
        return tuple(_make_sc_agg(True)(xs, ei))
    return tuple(_make_sc_agg(False)(xs, ei))


def kernel(x, edge_index, batch, W0, b0, W1l, b1l, W1r, W2l, b2l, W2r,
           We0, be0, We1, be1, We2, be2, Wm, bm, Wg, bg, Wv, bv, Wo, bo):
    src = edge_index[0]
    dst = edge_index[1]
    pad_i = jnp.arange(EP - E, dtype=jnp.int32)
    src2d = jnp.concatenate([src, pad_i % N]).reshape(EROWS, 128)
    dst2d = jnp.concatenate([dst, N + pad_i % (ACC_ROWS - N)]).reshape(EROWS, 128)
    ei = jnp.stack([src2d, dst2d], axis=1)

    x0s = _dense0(x, W0, b0)
    agg1, degp = _sage_agg(x0s.reshape(2, ACC_ROWS, HF), ei, True)
    dinvp = jnp.repeat(1.0 / jnp.maximum(degp, 1.0), HF).reshape(PK, 128)
    x1s = _dense_sage(agg1, dinvp, x0s, W1l, b1l, W1r)
    (agg2,) = _sage_agg(x1s.reshape(2, ACC_ROWS, HF), ei, False)
    x2s = _dense_sage(agg2, dinvp, x1s, W2l, b2l, W2r)
    return _final(x0s, x1s, x2s, batch, We0, be0, We1, be1, We2, be2,
                  Wm, bm, Wg, bg, Wv, bv, Wo, bo)


# speedup vs baseline: 13.9940x; 1.0815x over previous
import functools

import jax
import jax.numpy as jnp
from jax import lax
from jax.experimental import pallas as pl
from jax.experimental.pallas import tpu as pltpu
from jax.experimental.pallas import tpu_sc as plsc

N = 50000
E = 800000
NG = 256
DIN = 128
DH = 64
HF = 32
GRPH = 64

EP = 819200
EROWS = EP // 128
NC = 2
NS = 16
RPT = EROWS // (NS)
K = 2
CHUNKS = RPT // K
TRIPS = 66
SACC = 50176
SAPT = SACC // NS
ACC_ROWS = 51200
ZROWS = 64
APT = ACC_ROWS // NS

RB = 2000
GB = N // RB
PB = 512
PK = ACC_ROWS // 4
PN = N // 4
PKV = SACC // 4


def _bd4(W):
    return jnp.kron(jnp.eye(4, dtype=W.dtype), W)


def _t4(b):
    return jnp.tile(b, 4).reshape(1, -1)


def _mm0_body(x_ref, wa_ref, wb_ref, ba_ref, bb_ref, o_ref):
    xb = x_ref[...]
    o_ref[0] = jnp.dot(xb, wa_ref[...],
                       preferred_element_type=jnp.float32) + ba_ref[...]
    o_ref[1] = jnp.dot(xb, wb_ref[...],
                       preferred_element_type=jnp.float32) + bb_ref[...]


def _dense0(x, W0, b0):
    xp = jnp.pad(x.reshape(PN, 4 * DIN), ((0, PK - PN), (0, 0)))
    wspec = pl.BlockSpec((4 * DIN, 128), lambda i: (0, 0))
    bspec = pl.BlockSpec((1, 128), lambda i: (0, 0))
    return pl.pallas_call(
        _mm0_body,
        grid=(GB,),
        in_specs=[
            pl.BlockSpec((PB, 4 * DIN), lambda i: (i, 0)),
            wspec, wspec, bspec, bspec,
        ],
        out_specs=pl.BlockSpec((2, PB, 128), lambda i: (0, i, 0)),
        out_shape=jax.ShapeDtypeStruct((2, PK, 128), jnp.float32),
    )(xp, _bd4(W0[:, :HF]), _bd4(W0[:, HF:]), _t4(b0[:HF]), _t4(b0[HF:]))


def _sage_body(aa_ref, ab_ref, dinv_ref, xa_ref, xb_ref,
               wlaa, wlba, wraa, wrba, wlab, wlbb, wrab, wrbb,
               ba_ref, bb_ref, o_ref):
    def mm(a, b):
        return jnp.dot(a, b[...], preferred_element_type=jnp.float32)
    na = aa_ref[0] * dinv_ref[...]
    nb = ab_ref[0] * dinv_ref[...]
    xa = xa_ref[0]
    xb = xb_ref[0]
    yA = (mm(na, wlaa) + mm(nb, wlba) + mm(xa, wraa) + mm(xb, wrba)
          + ba_ref[...])
    yB = (mm(na, wlab) + mm(nb, wlbb) + mm(xa, wrab) + mm(xb, wrbb)
          + bb_ref[...])
    rid = pl.program_id(0) * PB + lax.broadcasted_iota(jnp.int32, (PB, 1), 0)
    mask = rid < PKV
    o_ref[0] = jnp.where(mask, yA, 0.0)
    o_ref[1] = jnp.where(mask, yB, 0.0)


def _dense_sage(aggs, dinvp, xs, Wl, bl, Wr):
    half = pl.BlockSpec((1, PB, 128), lambda i: (0, i, 0))
    half2 = pl.BlockSpec((1, PB, 128), lambda i: (1, i, 0))
    wspec = pl.BlockSpec((128, 128), lambda i: (0, 0))
    bspec = pl.BlockSpec((1, 128), lambda i: (0, 0))
    aggp = aggs.reshape(2, PK, 128)
    ws = [_bd4(Wl[:HF, :HF]), _bd4(Wl[HF:, :HF]),
          _bd4(Wr[:HF, :HF]), _bd4(Wr[HF:, :HF]),
          _bd4(Wl[:HF, HF:]), _bd4(Wl[HF:, HF:]),
          _bd4(Wr[:HF, HF:]), _bd4(Wr[HF:, HF:])]
    return pl.pallas_call(
        _sage_body,
        grid=(GB,),
        in_specs=[half, half2, pl.BlockSpec((PB, 128), lambda i: (i, 0)),
                  half, half2] + [wspec] * 8 + [bspec, bspec],
        out_specs=pl.BlockSpec((2, PB, 128), lambda i: (0, i, 0)),
        out_shape=jax.ShapeDtypeStruct((2, PK, 128), jnp.float32),
    )(aggp, aggp, dinvp, xs, xs, *ws, _t4(bl[:HF]), _t4(bl[HF:]))


def _make_sc_agg(with_deg):
    mesh = plsc.VectorSubcoreMesh(core_axis_name="c", subcore_axis_name="s")
    out_type = [jax.ShapeDtypeStruct((2, ACC_ROWS, HF), jnp.float32)]
    scratch = [
        pltpu.VMEM((K, 2, 128), jnp.int32),
        pltpu.VMEM((K, 2, 128), jnp.int32),
        pltpu.VMEM((K, 2, 128), jnp.int32),
        pltpu.VMEM((K * 128, HF), jnp.float32),
        pltpu.VMEM((K * 128, HF), jnp.float32),
        pltpu.VMEM((K * 128, HF), jnp.float32),
        pltpu.VMEM_SHARED((SACC, HF), jnp.float32),
        pltpu.SemaphoreType.DMA,
        pltpu.SemaphoreType.DMA,
        pltpu.SemaphoreType.DMA,
        pltpu.SemaphoreType.DMA,
        pltpu.SemaphoreType.DMA,
        pltpu.SemaphoreType.DMA,
        pltpu.SemaphoreType.DMA,
        pltpu.SemaphoreType.DMA,
        pltpu.SemaphoreType.DMA,
    ]
    if with_deg:
        out_type.append(jax.ShapeDtypeStruct((SACC,), jnp.float32))
        scratch += [
            pltpu.VMEM((448,), jnp.float32),
            pltpu.VMEM((128,), jnp.float32),
            pltpu.VMEM_SHARED((SACC,), jnp.float32),
        ]

    def body(tbl_ref, ei_ref, out_ref, *rest):
        if with_deg:
            (deg_out, sd0, sd1, sd2, rows0, rows1, rows2, acc,
             isem0, isem1, isem2, gsem0, gsem1, gsem2, ssem0, ssem1, ssem2,
             zbuf1, ones, dacc) = rest
        else:
            (sd0, sd1, sd2, rows0, rows1, rows2, acc,
             isem0, isem1, isem2, gsem0, gsem1, gsem2, ssem0, ssem1, ssem2) = rest
        c = lax.axis_index("c")
        s = lax.axis_index("s")
        mytbl = tbl_ref.at[c]
        myout = out_ref.at[c]
        sds = [sd0, sd1, sd2]
        rows = [rows0, rows1, rows2]
        isems = [isem0, isem1, isem2]
        gsems = [gsem0, gsem1, gsem2]
        ssems = [ssem0, ssem1, ssem2]

        zero16 = jnp.zeros((16,), jnp.float32)

        @pl.loop(0, K * 128)
        def _zfill(r):
            rows0[r, pl.ds(0, 16)] = zero16
            rows0[r, pl.ds(16, 16)] = zero16

        NZ = K * 128
        @pl.loop(0, SAPT // NZ)
        def _zacc(r):
            pltpu.sync_copy(rows0, acc.at[pl.ds(s * SAPT + r * NZ, NZ)])
        pltpu.sync_copy(rows0.at[pl.ds(0, SAPT % NZ)],
                        acc.at[pl.ds(s * SAPT + (SAPT // NZ) * NZ, SAPT % NZ)])

        if with_deg:
            @pl.when(c == 0)
            def _zdeg():
                @pl.loop(0, 28)
                def _zf1(r):
                    zbuf1[pl.ds(r * 16, 16)] = zero16
                one16 = jnp.ones((16,), jnp.float32)

                @pl.loop(0, 8)
                def _of(r):
                    ones[pl.ds(r * 16, 16)] = one16
                @pl.loop(0, 7)
                def _zd(r):
                    pltpu.sync_copy(zbuf1, dacc.at[pl.ds(s * SAPT + r * 448, 448)])

        plsc.subcore_barrier()

        def fire_idx(chunk_base, q):
            return pltpu.async_copy(ei_ref.at[pl.ds(chunk_base, K)], sds[q],
                                    isems[q])

        def fire_gathers(q):
            for j in range(K):
                pltpu.async_copy(mytbl.at[sds[q].at[j, 0]],
                                 rows[q].at[pl.ds(j * 128, 128)], gsems[q])

        def wait_gathers(q):
            for j in range(K):
                pltpu.make_async_copy(mytbl.at[sds[q].at[j, 0]],
                                      rows[q].at[pl.ds(j * 128, 128)],
                                      gsems[q]).wait()

        def fire_scatters(q):
            for j in range(K):
                pltpu.async_copy(rows[q].at[pl.ds(j * 128, 128)],
                                 acc.at[sds[q].at[j, 1]], ssems[q], add=True)
            if with_deg:
                @pl.when(c == 0)
                def _dfire():
                    for j in range(K):
                        pltpu.async_copy(ones, dacc.at[sds[q].at[j, 1]],
                                        ssems[q], add=True)

        def wait_scatters(q):
            for j in range(K):
                pltpu.make_async_copy(rows[q].at[pl.ds(j * 128, 128)],
                                      acc.at[sds[q].at[j, 1]], ssems[q]).wait()
            if with_deg:
                @pl.when(c == 0)
                def _dwait():
                    for j in range(K):
                        pltpu.make_async_copy(ones, dacc.at[sds[q].at[j, 1]],
                                              ssems[q]).wait()

        tbase = s * RPT
        for q in range(3):
            fire_idx(tbase + q * K, q).wait()
            fire_gathers(q)

        @pl.loop(0, TRIPS)
        def _trip(h):
            base = tbase + (3 * h) * K
            for q in range(3):
                wait_gathers(q)
                fire_scatters(q)
            for q in range(3):
                wait_scatters(q)
                if q < 2:
                    fire_idx(base + (q + 3) * K, q).wait()
                    fire_gathers(q)
                else:
                    @pl.when(h < TRIPS - 1)
                    def _nextC():
                        fire_idx(base + 5 * K, 2).wait()
                        fire_gathers(2)

        for q in range(2):
            wait_gathers(q)
            fire_scatters(q)
        for q in range(2):
            wait_scatters(q)

        plsc.subcore_barrier()

        pltpu.sync_copy(acc.at[pl.ds(s * SAPT, SAPT)],
                        myout.at[pl.ds(s * SAPT, SAPT)])
        if with_deg:
            @pl.when(c == 0)
            def _wdeg():
                pltpu.sync_copy(dacc.at[pl.ds(s * SAPT, SAPT)],
                                deg_out.at[pl.ds(s * SAPT, SAPT)])

    return pl.kernel(body, out_type=out_type, mesh=mesh, scratch_types=scratch,
                     compiler_params=pltpu.CompilerParams(
                         use_tc_tiling_on_sc=False))


_make_sc_agg = functools.cache(_make_sc_agg)


def _final_body(x0a, x0b, x1a, x1b, x2a, x2b, bat_ref,
                we0t, we0b, be0, we1t, we1b, be1, we2t, we2b, be2,
                wm0, wm1, wm2, bm, wg, bg, wv, bv, wo, bo,
                out_ref, m_ref, gsum_ref, pooled_ref):
    i = pl.program_id(0)

    @pl.when(i == 0)
    def _init():
        m_ref[...] = jnp.full((NG, 1), -1e30, jnp.float32)
        gsum_ref[...] = jnp.zeros((NG, 1), jnp.float32)
        pooled_ref[...] = jnp.zeros((NG, GRPH), jnp.float32)

    def mm(a, b):
        return jnp.dot(a, b, preferred_element_type=jnp.float32)

    e0 = jax.nn.relu(mm(x0a[0], we0t[...]) + mm(x0b[0], we0b[...]) + be0[...])
    e1 = jax.nn.relu(mm(x1a[0], we1t[...]) + mm(x1b[0], we1b[...]) + be1[...])
    e2 = jax.nn.relu(mm(x2a[0], we2t[...]) + mm(x2b[0], we2b[...]) + be2[...])
    xc = jax.nn.relu(mm(e0, wm0[...]) + mm(e1, wm1[...]) + mm(e2, wm2[...])
                     + bm[...])
    v = mm(xc, wv[...]) + bv[...]

    gids = lax.broadcasted_iota(jnp.int32, (NG, PB), 0)
    m_old = m_ref[...]
    gates, onehots, bmaxs = [], [], []
    for k in range(4):
        xck = xc[:, k * GRPH:(k + 1) * GRPH]
        gate_k = lax.dot_general(wg[...], xck, (((0,), (1,)), ((), ())),
                                 preferred_element_type=jnp.float32) + bg[...]
        b_k = bat_ref[0, pl.ds(k, 1)]
        eq_k = gids == b_k
        gates.append(gate_k)
        onehots.append(eq_k.astype(jnp.float32))
        bmaxs.append(jnp.max(jnp.where(eq_k, gate_k, -1e30), axis=1,
                             keepdims=True))
    m_new = jnp.maximum(jnp.maximum(jnp.maximum(m_old, bmaxs[0]),
                                    jnp.maximum(bmaxs[1], bmaxs[2])),
                        bmaxs[3])
    scale = jnp.exp(m_old - m_new)
    gsum_inc = jnp.zeros((NG, 1), jnp.float32)
    pooled_inc = jnp.zeros((NG, GRPH), jnp.float32)
    for k in range(4):
        m_node = lax.dot_general(m_new, onehots[k], (((0,), (0,)), ((), ())),
                                 preferred_element_type=jnp.float32)
        wmat = jnp.where(onehots[k] > 0.0,
                         jnp.exp(gates[k] - m_node), 0.0)
        gsum_inc = gsum_inc + jnp.sum(wmat, axis=1, keepdims=True)
        pooled_inc = pooled_inc + mm(wmat, v[:, k * GRPH:(k + 1) * GRPH])
    m_ref[...] = m_new
    gsum_ref[...] = gsum_ref[...] * scale + gsum_inc
    pooled_ref[...] = pooled_ref[...] * scale + pooled_inc

    @pl.when(i == GB - 1)
    def _fin():
        pooled = pooled_ref[...] / (gsum_ref[...] + 1e-16)
        logits = mm(pooled, wo[...]) + bo[...]
        lane = lax.broadcasted_iota(jnp.int32, (NG, 128), 1)
        logits = jnp.where(lane < 2, logits, -1e30)
        mx = jnp.max(logits, axis=1, keepdims=True)
        p = jnp.exp(logits - mx)
        out_ref[...] = p / jnp.sum(p, axis=1, keepdims=True)


def _final(x0s, x1s, x2s, batch, We0, be0, We1, be1, We2, be2, Wm, bm,
           Wg, bg, Wv, bv, Wo, bo):
    half = pl.BlockSpec((1, PB, 128), lambda i: (0, i, 0))
    half2 = pl.BlockSpec((1, PB, 128), lambda i: (1, i, 0))
    wspec = pl.BlockSpec((128, 4 * GRPH), lambda i: (0, 0))
    bspec = pl.BlockSpec((1, 4 * GRPH), lambda i: (0, 0))
    mspec = pl.BlockSpec((4 * DH, 4 * GRPH), lambda i: (0, 0))
    Wo128 = jnp.pad(Wo, ((0, 0), (0, 128 - Wo.shape[1])))
    bo128 = jnp.pad(bo.reshape(1, -1), ((0, 0), (0, 128 - bo.shape[0])))
    batp = jnp.concatenate(
        [batch, jnp.full((4 * PK - N,), -1, jnp.int32)]
    ).reshape(GB, PB, 4).transpose(0, 2, 1)
    out = pl.pallas_call(
        _final_body,
        grid=(GB,),
        in_specs=[
            half, half2, half, half2, half, half2,
            pl.BlockSpec((1, 4, PB), lambda i: (i, 0, 0)),
            wspec, wspec, bspec, wspec, wspec, bspec, wspec, wspec, bspec,
            mspec, mspec, mspec, bspec,
            pl.BlockSpec((DH, 1), lambda i: (0, 0)),
            pl.BlockSpec((1, 1), lambda i: (0, 0)),
            mspec,
            bspec,
            pl.BlockSpec((DH, 128), lambda i: (0, 0)),
            pl.BlockSpec((1, 128), lambda i: (0, 0)),
        ],
        out_specs=pl.BlockSpec((NG, 128), lambda i: (0, 0)),
        out_shape=jax.ShapeDtypeStruct((NG, 128), jnp.float32),
        compiler_params=pltpu.CompilerParams(
            dimension_semantics=("arbitrary",)),
        scratch_shapes=[
            pltpu.VMEM((NG, 1), jnp.float32),
            pltpu.VMEM((NG, 1), jnp.float32),
            pltpu.VMEM((NG, GRPH), jnp.float32),
        ],
    )(x0s, x0s, x1s, x1s, x2s, x2s, batp,
      _bd4(We0[:HF]), _bd4(We0[HF:]), _t4(be0),
      _bd4(We1[:HF]), _bd4(We1[HF:]), _t4(be1),
      _bd4(We2[:HF]), _bd4(We2[HF:]), _t4(be2),
      _bd4(Wm[:DH]), _bd4(Wm[DH:2 * DH]), _bd4(Wm[2 * DH:]), _t4(bm),
      Wg, bg.reshape(1, 1), _bd4(Wv), _t4(bv), Wo128, bo128)
    return out[:, :2]


def _sage_agg(xs, ei, with_deg):
    if with_deg:
        return tuple(_make_sc_agg(True)(xs, ei))
    return tuple(_make_sc_agg(False)(xs, ei))


def kernel(x, edge_index, batch, W0, b0, W1l, b1l, W1r, W2l, b2l, W2r,
           We0, be0, We1, be1, We2, be2, Wm, bm, Wg, bg, Wv, bv, Wo, bo):
    src = edge_index[0]
    dst = edge_index[1]
    pad_i = jnp.arange(EP - E, dtype=jnp.int32)
    src2d = jnp.concatenate([src, pad_i % N]).reshape(EROWS, 128)
    dst2d = jnp.concatenate([dst, N + pad_i % (SACC - N)]).reshape(EROWS, 128)
    ei = jnp.stack([src2d, dst2d], axis=1)

    x0s = _dense0(x, W0, b0)
    agg1, degp = _sage_agg(x0s.reshape(2, ACC_ROWS, HF), ei, True)
    degf = jnp.pad(degp, (0, 4 * PK - SACC))
    dinvp = jnp.repeat(1.0 / jnp.maximum(degf, 1.0), HF).reshape(PK, 128)
    x1s = _dense_sage(agg1, dinvp, x0s, W1l, b1l, W1r)
    (agg2,) = _sage_agg(x1s.reshape(2, ACC_ROWS, HF), ei, False)
    x2s = _dense_sage(agg2, dinvp, x1s, W2l, b2l, W2r)
    return _final(x0s, x1s, x2s, batch, We0, be0, We1, be1, We2, be2,
                  Wm, bm, Wg, bg, Wv, bv, Wo, bo)


# speedup vs baseline: 14.6557x; 1.0473x over previous
import functools

import jax
import jax.numpy as jnp
from jax import lax
from jax.experimental import pallas as pl
from jax.experimental.pallas import tpu as pltpu
from jax.experimental.pallas import tpu_sc as plsc

N = 50000
E = 800000
NG = 256
DIN = 128
DH = 64
HF = 32
GRPH = 64

EP = 819200
EROWS = EP // 128
NC = 2
NS = 16
RPT = EROWS // (NS)
K = 2
CHUNKS = RPT // K
TRIPS = 66
SACC = 50176
SAPT = SACC // NS
ACC_ROWS = 51200
ZROWS = 64
APT = ACC_ROWS // NS

RB = 2000
GB = N // RB
PB = 512
PK = ACC_ROWS // 4
PN = N // 4
PKV = SACC // 4


def _bd4(W):
    return jnp.kron(jnp.eye(4, dtype=W.dtype), W)


def _t4(b):
    return jnp.tile(b, 4).reshape(1, -1)


def _mm0_body(x_ref, wa_ref, wb_ref, ba_ref, bb_ref, o_ref):
    xb = x_ref[...]
    o_ref[0] = jnp.dot(xb, wa_ref[...],
                       preferred_element_type=jnp.float32) + ba_ref[...]
    o_ref[1] = jnp.dot(xb, wb_ref[...],
                       preferred_element_type=jnp.float32) + bb_ref[...]


def _dense0(x, W0, b0):
    xp = x.reshape(PN, 4 * DIN)
    wspec = pl.BlockSpec((4 * DIN, 128), lambda i: (0, 0))
    bspec = pl.BlockSpec((1, 128), lambda i: (0, 0))
    return pl.pallas_call(
        _mm0_body,
        grid=(GB,),
        in_specs=[
            pl.BlockSpec((PB, 4 * DIN), lambda i: (i, 0)),
            wspec, wspec, bspec, bspec,
        ],
        out_specs=pl.BlockSpec((2, PB, 128), lambda i: (0, i, 0)),
        out_shape=jax.ShapeDtypeStruct((2, PK, 128), jnp.float32),
    )(xp, _bd4(W0[:, :HF]), _bd4(W0[:, HF:]), _t4(b0[:HF]), _t4(b0[HF:]))


def _sage_body(aa_ref, ab_ref, dq_ref, sel_ref, xa_ref, xb_ref,
               wlaa, wlba, wraa, wrba, wlab, wlbb, wrab, wrbb,
               ba_ref, bb_ref, o_ref):
    def mm(a, b):
        return jnp.dot(a, b[...], preferred_element_type=jnp.float32)
    dpk = jnp.dot(dq_ref[...], sel_ref[...],
                  preferred_element_type=jnp.float32)
    inv = 1.0 / jnp.maximum(dpk, 1.0)
    na = aa_ref[0] * inv
    nb = ab_ref[0] * inv
    xa = xa_ref[0]
    xb = xb_ref[0]
    o_ref[0] = (mm(na, wlaa) + mm(nb, wlba) + mm(xa, wraa) + mm(xb, wrba)
                + ba_ref[...])
    o_ref[1] = (mm(na, wlab) + mm(nb, wlbb) + mm(xa, wrab) + mm(xb, wrbb)
                + bb_ref[...])


def _dense_sage(aggs, degq, sel, xs, Wl, bl, Wr):
    half = pl.BlockSpec((1, PB, 128), lambda i: (0, i, 0))
    half2 = pl.BlockSpec((1, PB, 128), lambda i: (1, i, 0))
    wspec = pl.BlockSpec((128, 128), lambda i: (0, 0))
    bspec = pl.BlockSpec((1, 128), lambda i: (0, 0))
    aggp = aggs.reshape(2, PK, 128)
    ws = [_bd4(Wl[:HF, :HF]), _bd4(Wl[HF:, :HF]),
          _bd4(Wr[:HF, :HF]), _bd4(Wr[HF:, :HF]),
          _bd4(Wl[:HF, HF:]), _bd4(Wl[HF:, HF:]),
          _bd4(Wr[:HF, HF:]), _bd4(Wr[HF:, HF:])]
    return pl.pallas_call(
        _sage_body,
        grid=(GB,),
        in_specs=[half, half2, pl.BlockSpec((PB, 4), lambda i: (i, 0)),
                  pl.BlockSpec((4, 128), lambda i: (0, 0)),
                  half, half2] + [wspec] * 8 + [bspec, bspec],
        out_specs=pl.BlockSpec((2, PB, 128), lambda i: (0, i, 0)),
        out_shape=jax.ShapeDtypeStruct((2, PK, 128), jnp.float32),
    )(aggp, aggp, degq, sel, xs, xs, *ws, _t4(bl[:HF]), _t4(bl[HF:]))


def _make_sc_agg(with_deg):
    mesh = plsc.VectorSubcoreMesh(core_axis_name="c", subcore_axis_name="s")
    out_type = [jax.ShapeDtypeStruct((2, ACC_ROWS, HF), jnp.float32)]
    scratch = [
        pltpu.VMEM((K, 2, 128), jnp.int32),
        pltpu.VMEM((K, 2, 128), jnp.int32),
        pltpu.VMEM((K, 2, 128), jnp.int32),
        pltpu.VMEM((K * 128, HF), jnp.float32),
        pltpu.VMEM((K * 128, HF), jnp.float32),
        pltpu.VMEM((K * 128, HF), jnp.float32),
        pltpu.VMEM_SHARED((SACC, HF), jnp.float32),
        pltpu.SemaphoreType.DMA,
        pltpu.SemaphoreType.DMA,
        pltpu.SemaphoreType.DMA,
        pltpu.SemaphoreType.DMA,
        pltpu.SemaphoreType.DMA,
        pltpu.SemaphoreType.DMA,
        pltpu.SemaphoreType.DMA,
        pltpu.SemaphoreType.DMA,
        pltpu.SemaphoreType.DMA,
    ]
    if with_deg:
        out_type.append(jax.ShapeDtypeStruct((SACC,), jnp.float32))
        scratch += [
            pltpu.VMEM((448,), jnp.float32),
            pltpu.VMEM((128,), jnp.float32),
            pltpu.VMEM_SHARED((SACC,), jnp.float32),
        ]

    def body(tbl_ref, ei_ref, out_ref, *rest):
        if with_deg:
            (deg_out, sd0, sd1, sd2, rows0, rows1, rows2, acc,
             isem0, isem1, isem2, gsem0, gsem1, gsem2, ssem0, ssem1, ssem2,
             zbuf1, ones, dacc) = rest
        else:
            (sd0, sd1, sd2, rows0, rows1, rows2, acc,
             isem0, isem1, isem2, gsem0, gsem1, gsem2, ssem0, ssem1, ssem2) = rest
        c = lax.axis_index("c")
        s = lax.axis_index("s")
        mytbl = tbl_ref.at[c]
        myout = out_ref.at[c]
        sds = [sd0, sd1, sd2]
        rows = [rows0, rows1, rows2]
        isems = [isem0, isem1, isem2]
        gsems = [gsem0, gsem1, gsem2]
        ssems = [ssem0, ssem1, ssem2]

        zero16 = jnp.zeros((16,), jnp.float32)

        @pl.loop(0, K * 128)
        def _zfill(r):
            rows0[r, pl.ds(0, 16)] = zero16
            rows0[r, pl.ds(16, 16)] = zero16

        NZ = K * 128
        @pl.loop(0, SAPT // NZ)
        def _zacc(r):
            pltpu.sync_copy(rows0, acc.at[pl.ds(s * SAPT + r * NZ, NZ)])
        pltpu.sync_copy(rows0.at[pl.ds(0, SAPT % NZ)],
                        acc.at[pl.ds(s * SAPT + (SAPT // NZ) * NZ, SAPT % NZ)])

        if with_deg:
            @pl.when(c == 0)
            def _zdeg():
                @pl.loop(0, 28)
                def _zf1(r):
                    zbuf1[pl.ds(r * 16, 16)] = zero16
                one16 = jnp.ones((16,), jnp.float32)

                @pl.loop(0, 8)
                def _of(r):
                    ones[pl.ds(r * 16, 16)] = one16
                @pl.loop(0, 7)
                def _zd(r):
                    pltpu.sync_copy(zbuf1, dacc.at[pl.ds(s * SAPT + r * 448, 448)])

        plsc.subcore_barrier()

        def fire_idx(chunk_base, q):
            return pltpu.async_copy(ei_ref.at[pl.ds(chunk_base, K)], sds[q],
                                    isems[q])

        def fire_gathers(q):
            for j in range(K):
                pltpu.async_copy(mytbl.at[sds[q].at[j, 0]],
                                 rows[q].at[pl.ds(j * 128, 128)], gsems[q])

        def wait_gathers(q):
            for j in range(K):
                pltpu.make_async_copy(mytbl.at[sds[q].at[j, 0]],
                                      rows[q].at[pl.ds(j * 128, 128)],
                                      gsems[q]).wait()

        def fire_scatters(q):
            for j in range(K):
                pltpu.async_copy(rows[q].at[pl.ds(j * 128, 128)],
                                 acc.at[sds[q].at[j, 1]], ssems[q], add=True)
            if with_deg:
                @pl.when(c == 0)
                def _dfire():
                    for j in range(K):
                        pltpu.async_copy(ones, dacc.at[sds[q].at[j, 1]],
                                        ssems[q], add=True)

        def wait_scatters(q):
            for j in range(K):
                pltpu.make_async_copy(rows[q].at[pl.ds(j * 128, 128)],
                                      acc.at[sds[q].at[j, 1]], ssems[q]).wait()
            if with_deg:
                @pl.when(c == 0)
                def _dwait():
                    for j in range(K):
                        pltpu.make_async_copy(ones, dacc.at[sds[q].at[j, 1]],
                                              ssems[q]).wait()

        tbase = s * RPT
        for q in range(3):
            fire_idx(tbase + q * K, q).wait()
            fire_gathers(q)

        @pl.loop(0, TRIPS)
        def _trip(h):
            base = tbase + (3 * h) * K
            for q in range(3):
                wait_gathers(q)
                fire_scatters(q)
            for q in range(3):
                wait_scatters(q)
                if q < 2:
                    fire_idx(base + (q + 3) * K, q).wait()
                    fire_gathers(q)
                else:
                    @pl.when(h < TRIPS - 1)
                    def _nextC():
                        fire_idx(base + 5 * K, 2).wait()
                        fire_gathers(2)

        for q in range(2):
            wait_gathers(q)
            fire_scatters(q)
        for q in range(2):
            wait_scatters(q)

        plsc.subcore_barrier()

        pltpu.sync_copy(acc.at[pl.ds(s * SAPT, SAPT)],
                        myout.at[pl.ds(s * SAPT, SAPT)])
        if with_deg:
            @pl.when(c == 0)
            def _wdeg():
                pltpu.sync_copy(dacc.at[pl.ds(s * SAPT, SAPT)],
                                deg_out.at[pl.ds(s * SAPT, SAPT)])

    return pl.kernel(body, out_type=out_type, mesh=mesh, scratch_types=scratch,
                     compiler_params=pltpu.CompilerParams(
                         use_tc_tiling_on_sc=False))


_make_sc_agg = functools.cache(_make_sc_agg)


def _final_body(x0a, x0b, x1a, x1b, x2a, x2b, bat_ref,
                we0t, we0b, be0, we1t, we1b, be1, we2t, we2b, be2,
                wm0, wm1, wm2, bm, wg, bg, wv, bv, wo, bo,
                out_ref, m_ref, gsum_ref, pooled_ref):
    i = pl.program_id(0)

    @pl.when(i == 0)
    def _init():
        m_ref[...] = jnp.full((NG, 1), -1e30, jnp.float32)
        gsum_ref[...] = jnp.zeros((NG, 1), jnp.float32)
        pooled_ref[...] = jnp.zeros((NG, GRPH), jnp.float32)

    def mm(a, b):
        return jnp.dot(a, b, preferred_element_type=jnp.float32)

    e0 = jax.nn.relu(mm(x0a[0], we0t[...]) + mm(x0b[0], we0b[...]) + be0[...])
    e1 = jax.nn.relu(mm(x1a[0], we1t[...]) + mm(x1b[0], we1b[...]) + be1[...])
    e2 = jax.nn.relu(mm(x2a[0], we2t[...]) + mm(x2b[0], we2b[...]) + be2[...])
    xc = jax.nn.relu(mm(e0, wm0[...]) + mm(e1, wm1[...]) + mm(e2, wm2[...])
                     + bm[...])
    v = mm(xc, wv[...]) + bv[...]
    rid = i * PB + lax.broadcasted_iota(jnp.int32, (PB, 1), 0)
    v = jnp.where(rid < PN, v, 0.0)

    gids = lax.broadcasted_iota(jnp.int32, (NG, PB), 0)
    m_old = m_ref[...]
    gates, onehots, bmaxs = [], [], []
    for k in range(4):
        xck = xc[:, k * GRPH:(k + 1) * GRPH]
        gate_k = lax.dot_general(wg[...], xck, (((0,), (1,)), ((), ())),
                                 preferred_element_type=jnp.float32) + bg[...]
        b_k = bat_ref[0, pl.ds(k, 1)]
        eq_k = gids == b_k
        gates.append(gate_k)
        onehots.append(eq_k.astype(jnp.float32))
        bmaxs.append(jnp.max(jnp.where(eq_k, gate_k, -1e30), axis=1,
                             keepdims=True))
    m_new = jnp.maximum(jnp.maximum(jnp.maximum(m_old, bmaxs[0]),
                                    jnp.maximum(bmaxs[1], bmaxs[2])),
                        bmaxs[3])
    scale = jnp.exp(m_old - m_new)
    gsum_inc = jnp.zeros((NG, 1), jnp.float32)
    pooled_inc = jnp.zeros((NG, GRPH), jnp.float32)
    for k in range(4):
        m_node = lax.dot_general(m_new, onehots[k], (((0,), (0,)), ((), ())),
                                 preferred_element_type=jnp.float32)
        wmat = jnp.where(onehots[k] > 0.0,
                         jnp.exp(gates[k] - m_node), 0.0)
        gsum_inc = gsum_inc + jnp.sum(wmat, axis=1, keepdims=True)
        pooled_inc = pooled_inc + mm(wmat, v[:, k * GRPH:(k + 1) * GRPH])
    m_ref[...] = m_new
    gsum_ref[...] = gsum_ref[...] * scale + gsum_inc
    pooled_ref[...] = pooled_ref[...] * scale + pooled_inc

    @pl.when(i == GB - 1)
    def _fin():
        pooled = pooled_ref[...] / (gsum_ref[...] + 1e-16)
        logits = mm(pooled, wo[...]) + bo[...]
        lane = lax.broadcasted_iota(jnp.int32, (NG, 128), 1)
        logits = jnp.where(lane < 2, logits, -1e30)
        mx = jnp.max(logits, axis=1, keepdims=True)
        p = jnp.exp(logits - mx)
        out_ref[...] = p / jnp.sum(p, axis=1, keepdims=True)


def _final(x0s, x1s, x2s, batch, We0, be0, We1, be1, We2, be2, Wm, bm,
           Wg, bg, Wv, bv, Wo, bo):
    half = pl.BlockSpec((1, PB, 128), lambda i: (0, i, 0))
    half2 = pl.BlockSpec((1, PB, 128), lambda i: (1, i, 0))
    wspec = pl.BlockSpec((128, 4 * GRPH), lambda i: (0, 0))
    bspec = pl.BlockSpec((1, 4 * GRPH), lambda i: (0, 0))
    mspec = pl.BlockSpec((4 * DH, 4 * GRPH), lambda i: (0, 0))
    Wo128 = jnp.pad(Wo, ((0, 0), (0, 128 - Wo.shape[1])))
    bo128 = jnp.pad(bo.reshape(1, -1), ((0, 0), (0, 128 - bo.shape[0])))
    batp = jnp.concatenate(
        [batch, jnp.full((4 * PK - N,), -1, jnp.int32)]
    ).reshape(GB, PB, 4).transpose(0, 2, 1)
    out = pl.pallas_call(
        _final_body,
        grid=(GB,),
        in_specs=[
            half, half2, half, half2, half, half2,
            pl.BlockSpec((1, 4, PB), lambda i: (i, 0, 0)),
            wspec, wspec, bspec, wspec, wspec, bspec, wspec, wspec, bspec,
            mspec, mspec, mspec, bspec,
            pl.BlockSpec((DH, 1), lambda i: (0, 0)),
            pl.BlockSpec((1, 1), lambda i: (0, 0)),
            mspec,
            bspec,
            pl.BlockSpec((DH, 128), lambda i: (0, 0)),
            pl.BlockSpec((1, 128), lambda i: (0, 0)),
        ],
        out_specs=pl.BlockSpec((NG, 128), lambda i: (0, 0)),
        out_shape=jax.ShapeDtypeStruct((NG, 128), jnp.float32),
        compiler_params=pltpu.CompilerParams(
            dimension_semantics=("arbitrary",)),
        scratch_shapes=[
            pltpu.VMEM((NG, 1), jnp.float32),
            pltpu.VMEM((NG, 1), jnp.float32),
            pltpu.VMEM((NG, GRPH), jnp.float32),
        ],
    )(x0s, x0s, x1s, x1s, x2s, x2s, batp,
      _bd4(We0[:HF]), _bd4(We0[HF:]), _t4(be0),
      _bd4(We1[:HF]), _bd4(We1[HF:]), _t4(be1),
      _bd4(We2[:HF]), _bd4(We2[HF:]), _t4(be2),
      _bd4(Wm[:DH]), _bd4(Wm[DH:2 * DH]), _bd4(Wm[2 * DH:]), _t4(bm),
      Wg, bg.reshape(1, 1), _bd4(Wv), _t4(bv), Wo128, bo128)
    return out[:, :2]


def _sage_agg(xs, ei, with_deg):
    if with_deg:
        return tuple(_make_sc_agg(True)(xs, ei))
    return tuple(_make_sc_agg(False)(xs, ei))


def kernel(x, edge_index, batch, W0, b0, W1l, b1l, W1r, W2l, b2l, W2r,
           We0, be0, We1, be1, We2, be2, Wm, bm, Wg, bg, Wv, bv, Wo, bo):
    src = edge_index[0]
    dst = edge_index[1]
    pad_i = jnp.arange(EP - E, dtype=jnp.int32)
    src2d = jnp.concatenate([src, pad_i % N]).reshape(EROWS, 128)
    dst2d = jnp.concatenate([dst, N + pad_i % (SACC - N)]).reshape(EROWS, 128)
    ei = jnp.stack([src2d, dst2d], axis=1)

    x0s = _dense0(x, W0, b0)
    agg1, degp = _sage_agg(x0s.reshape(2, ACC_ROWS, HF), ei, True)
    degq = jnp.pad(degp, (0, 4 * PK - SACC)).reshape(PK, 4)
    sel = jnp.kron(jnp.eye(4, dtype=jnp.float32), jnp.ones((1, HF), jnp.float32))
    x1s = _dense_sage(agg1, degq, sel, x0s, W1l, b1l, W1r)
    (agg2,) = _sage_agg(x1s.reshape(2, ACC_ROWS, HF), ei, False)
    x2s = _dense_sage(agg2, degq, sel, x1s, W2l, b2l, W2r)
    return _final(x0s, x1s, x2s, batch, We0, be0, We1, be1, We2, be2,
                  Wm, bm, Wg, bg, Wv, bv, Wo, bo)


# speedup vs baseline: 15.1750x; 1.0354x over previous
import functools

import jax
import jax.numpy as jnp
from jax import lax
from jax.experimental import pallas as pl
from jax.experimental.pallas import tpu as pltpu
from jax.experimental.pallas import tpu_sc as plsc

N = 50000
E = 800000
NG = 256
DIN = 128
DH = 64
HF = 32
GRPH = 64

EP = 819200
EROWS = EP // 128
NC = 2
NS = 16
RPT = EROWS // (NS)
K = 2
CHUNKS = RPT // K
TRIPS = 66
SACC = 50176
SAPT = SACC // NS
ACC_ROWS = 51200
ZROWS = 64
APT = ACC_ROWS // NS

RB = 2000
GB = N // RB
PB = 512
PK = ACC_ROWS // 4
PN = N // 4
PKV = SACC // 4


def _bd4(W):
    return jnp.kron(jnp.eye(4, dtype=W.dtype), W)


def _t4(b):
    return jnp.tile(b, 4).reshape(1, -1)


def _mm0_body(x_ref, wa_ref, wb_ref, ba_ref, bb_ref, o_ref):
    xb = x_ref[...]
    o_ref[0] = jnp.dot(xb, wa_ref[...],
                       preferred_element_type=jnp.float32) + ba_ref[...]
    o_ref[1] = jnp.dot(xb, wb_ref[...],
                       preferred_element_type=jnp.float32) + bb_ref[...]


def _dense0(x, W0, b0):
    xp = x.reshape(PN, 4 * DIN)
    wspec = pl.BlockSpec((4 * DIN, 128), lambda i: (0, 0))
    bspec = pl.BlockSpec((1, 128), lambda i: (0, 0))
    return pl.pallas_call(
        _mm0_body,
        grid=(GB,),
        in_specs=[
            pl.BlockSpec((PB, 4 * DIN), lambda i: (i, 0)),
            wspec, wspec, bspec, bspec,
        ],
        out_specs=pl.BlockSpec((2, PB, 128), lambda i: (0, i, 0)),
        out_shape=jax.ShapeDtypeStruct((2, PK, 128), jnp.float32),
    )(xp, _bd4(W0[:, :HF]), _bd4(W0[:, HF:]), _t4(b0[:HF]), _t4(b0[HF:]))


def _sage_body(aa_ref, ab_ref, dq_ref, sel_ref, xa_ref, xb_ref,
               wlaa, wlba, wraa, wrba, wlab, wlbb, wrab, wrbb,
               ba_ref, bb_ref, o_ref):
    def mm(a, b):
        return jnp.dot(a, b[...], preferred_element_type=jnp.float32)
    dpk = jnp.dot(dq_ref[...], sel_ref[...],
                  preferred_element_type=jnp.float32)
    inv = 1.0 / jnp.maximum(dpk, 1.0)
    na = aa_ref[0] * inv
    nb = ab_ref[0] * inv
    xa = xa_ref[0]
    xb = xb_ref[0]
    o_ref[0] = (mm(na, wlaa) + mm(nb, wlba) + mm(xa, wraa) + mm(xb, wrba)
                + ba_ref[...])
    o_ref[1] = (mm(na, wlab) + mm(nb, wlbb) + mm(xa, wrab) + mm(xb, wrbb)
                + bb_ref[...])


def _dense_sage(aggs, degq, sel, xs, Wl, bl, Wr):
    half = pl.BlockSpec((1, PB, 128), lambda i: (0, i, 0))
    half2 = pl.BlockSpec((1, PB, 128), lambda i: (1, i, 0))
    wspec = pl.BlockSpec((128, 128), lambda i: (0, 0))
    bspec = pl.BlockSpec((1, 128), lambda i: (0, 0))
    aggp = aggs.reshape(2, PK, 128)
    ws = [_bd4(Wl[:HF, :HF]), _bd4(Wl[HF:, :HF]),
          _bd4(Wr[:HF, :HF]), _bd4(Wr[HF:, :HF]),
          _bd4(Wl[:HF, HF:]), _bd4(Wl[HF:, HF:]),
          _bd4(Wr[:HF, HF:]), _bd4(Wr[HF:, HF:])]
    return pl.pallas_call(
        _sage_body,
        grid=(GB,),
        in_specs=[half, half2, pl.BlockSpec((PB, 4), lambda i: (i, 0)),
                  pl.BlockSpec((4, 128), lambda i: (0, 0)),
                  half, half2] + [wspec] * 8 + [bspec, bspec],
        out_specs=pl.BlockSpec((2, PB, 128), lambda i: (0, i, 0)),
        out_shape=jax.ShapeDtypeStruct((2, PK, 128), jnp.float32),
    )(aggp, aggp, degq, sel, xs, xs, *ws, _t4(bl[:HF]), _t4(bl[HF:]))


def _make_sc_agg(with_deg):
    mesh = plsc.VectorSubcoreMesh(core_axis_name="c", subcore_axis_name="s")
    out_type = [jax.ShapeDtypeStruct((2, ACC_ROWS, HF), jnp.float32)]
    scratch = [
        pltpu.VMEM((2, K, 128), jnp.int32),
        pltpu.VMEM((2, K, 128), jnp.int32),
        pltpu.VMEM((2, K, 128), jnp.int32),
        pltpu.VMEM((K * 128, HF), jnp.float32),
        pltpu.VMEM((K * 128, HF), jnp.float32),
        pltpu.VMEM((K * 128, HF), jnp.float32),
        pltpu.VMEM_SHARED((SACC, HF), jnp.float32),
        pltpu.SemaphoreType.DMA,
        pltpu.SemaphoreType.DMA,
        pltpu.SemaphoreType.DMA,
        pltpu.SemaphoreType.DMA,
        pltpu.SemaphoreType.DMA,
        pltpu.SemaphoreType.DMA,
        pltpu.SemaphoreType.DMA,
        pltpu.SemaphoreType.DMA,
        pltpu.SemaphoreType.DMA,
    ]
    if with_deg:
        out_type.append(jax.ShapeDtypeStruct((SACC,), jnp.float32))
        scratch += [
            pltpu.VMEM((448,), jnp.float32),
            pltpu.VMEM((128,), jnp.float32),
            pltpu.VMEM_SHARED((SACC,), jnp.float32),
        ]

    def body(tbl_ref, ei_ref, out_ref, *rest):
        if with_deg:
            (deg_out, sd0, sd1, sd2, rows0, rows1, rows2, acc,
             isem0, isem1, isem2, gsem0, gsem1, gsem2, ssem0, ssem1, ssem2,
             zbuf1, ones, dacc) = rest
        else:
            (sd0, sd1, sd2, rows0, rows1, rows2, acc,
             isem0, isem1, isem2, gsem0, gsem1, gsem2, ssem0, ssem1, ssem2) = rest
        c = lax.axis_index("c")
        s = lax.axis_index("s")
        mytbl = tbl_ref.at[c]
        myout = out_ref.at[c]
        sds = [sd0, sd1, sd2]
        rows = [rows0, rows1, rows2]
        isems = [isem0, isem1, isem2]
        gsems = [gsem0, gsem1, gsem2]
        ssems = [ssem0, ssem1, ssem2]

        zero16 = jnp.zeros((16,), jnp.float32)

        @pl.loop(0, K * 128)
        def _zfill(r):
            rows0[r, pl.ds(0, 16)] = zero16
            rows0[r, pl.ds(16, 16)] = zero16

        NZ = K * 128
        @pl.loop(0, SAPT // NZ)
        def _zacc(r):
            pltpu.sync_copy(rows0, acc.at[pl.ds(s * SAPT + r * NZ, NZ)])
        pltpu.sync_copy(rows0.at[pl.ds(0, SAPT % NZ)],
                        acc.at[pl.ds(s * SAPT + (SAPT // NZ) * NZ, SAPT % NZ)])

        if with_deg:
            @pl.when(c == 0)
            def _zdeg():
                @pl.loop(0, 28)
                def _zf1(r):
                    zbuf1[pl.ds(r * 16, 16)] = zero16
                one16 = jnp.ones((16,), jnp.float32)

                @pl.loop(0, 8)
                def _of(r):
                    ones[pl.ds(r * 16, 16)] = one16
                @pl.loop(0, 7)
                def _zd(r):
                    pltpu.sync_copy(zbuf1, dacc.at[pl.ds(s * SAPT + r * 448, 448)])

        plsc.subcore_barrier()

        def fire_idx(chunk_base, q):
            pltpu.async_copy(ei_ref.at[0, pl.ds(chunk_base, K)],
                             sds[q].at[0], isems[q])
            d = pltpu.async_copy(ei_ref.at[1, pl.ds(chunk_base, K)],
                                 sds[q].at[1], isems[q])
            class _W:
                def wait(self):
                    pltpu.make_async_copy(ei_ref.at[0, pl.ds(chunk_base, K)],
                                          sds[q].at[0], isems[q]).wait()
                    d.wait()
            return _W()

        def fire_gathers(q):
            for j in range(K):
                pltpu.async_copy(mytbl.at[sds[q].at[0, j]],
                                 rows[q].at[pl.ds(j * 128, 128)], gsems[q])

        def wait_gathers(q):
            for j in range(K):
                pltpu.make_async_copy(mytbl.at[sds[q].at[0, j]],
                                      rows[q].at[pl.ds(j * 128, 128)],
                                      gsems[q]).wait()

        def fire_scatters(q):
            for j in range(K):
                pltpu.async_copy(rows[q].at[pl.ds(j * 128, 128)],
                                 acc.at[sds[q].at[1, j]], ssems[q], add=True)
            if with_deg:
                @pl.when(c == 0)
                def _dfire():
                    for j in range(K):
                        pltpu.async_copy(ones, dacc.at[sds[q].at[1, j]],
                                        ssems[q], add=True)

        def wait_scatters(q):
            for j in range(K):
                pltpu.make_async_copy(rows[q].at[pl.ds(j * 128, 128)],
                                      acc.at[sds[q].at[1, j]], ssems[q]).wait()
            if with_deg:
                @pl.when(c == 0)
                def _dwait():
                    for j in range(K):
                        pltpu.make_async_copy(ones, dacc.at[sds[q].at[1, j]],
                                              ssems[q]).wait()

        tbase = s * RPT
        for q in range(3):
            fire_idx(tbase + q * K, q).wait()
            fire_gathers(q)

        @pl.loop(0, TRIPS)
        def _trip(h):
            base = tbase + (3 * h) * K
            for q in range(3):
                wait_gathers(q)
                fire_scatters(q)
            for q in range(3):
                wait_scatters(q)
                if q < 2:
                    fire_idx(base + (q + 3) * K, q).wait()
                    fire_gathers(q)
                else:
                    @pl.when(h < TRIPS - 1)
                    def _nextC():
                        fire_idx(base + 5 * K, 2).wait()
                        fire_gathers(2)

        for q in range(2):
            wait_gathers(q)
            fire_scatters(q)
        for q in range(2):
            wait_scatters(q)

        plsc.subcore_barrier()

        pltpu.sync_copy(acc.at[pl.ds(s * SAPT, SAPT)],
                        myout.at[pl.ds(s * SAPT, SAPT)])
        if with_deg:
            @pl.when(c == 0)
            def _wdeg():
                pltpu.sync_copy(dacc.at[pl.ds(s * SAPT, SAPT)],
                                deg_out.at[pl.ds(s * SAPT, SAPT)])

    return pl.kernel(body, out_type=out_type, mesh=mesh, scratch_types=scratch,
                     compiler_params=pltpu.CompilerParams(
                         use_tc_tiling_on_sc=False))


_make_sc_agg = functools.cache(_make_sc_agg)


def _final_body(x0a, x0b, x1a, x1b, x2a, x2b, bat_ref,
                we0t, we0b, be0, we1t, we1b, be1, we2t, we2b, be2,
                wm0, wm1, wm2, bm, wg, bg, wv, bv, wo, bo,
                out_ref, m_ref, gsum_ref, pooled_ref):
    i = pl.program_id(0)

    @pl.when(i == 0)
    def _init():
        m_ref[...] = jnp.full((NG, 1), -1e30, jnp.float32)
        gsum_ref[...] = jnp.zeros((NG, 1), jnp.float32)
        pooled_ref[...] = jnp.zeros((NG, GRPH), jnp.float32)

    def mm(a, b):
        return jnp.dot(a, b, preferred_element_type=jnp.float32)

    e0 = jax.nn.relu(mm(x0a[0], we0t[...]) + mm(x0b[0], we0b[...]) + be0[...])
    e1 = jax.nn.relu(mm(x1a[0], we1t[...]) + mm(x1b[0], we1b[...]) + be1[...])
    e2 = jax.nn.relu(mm(x2a[0], we2t[...]) + mm(x2b[0], we2b[...]) + be2[...])
    xc = jax.nn.relu(mm(e0, wm0[...]) + mm(e1, wm1[...]) + mm(e2, wm2[...])
                     + bm[...])
    v = mm(xc, wv[...]) + bv[...]
    rid = i * PB + lax.broadcasted_iota(jnp.int32, (PB, 1), 0)
    v = jnp.where(rid < PN, v, 0.0)

    gids = lax.broadcasted_iota(jnp.int32, (NG, PB), 0)
    m_old = m_ref[...]
    gates, onehots, bmaxs = [], [], []
    for k in range(4):
        xck = xc[:, k * GRPH:(k + 1) * GRPH]
        gate_k = lax.dot_general(wg[...], xck, (((0,), (1,)), ((), ())),
                                 preferred_element_type=jnp.float32) + bg[...]
        b_k = bat_ref[0, pl.ds(k, 1)]
        eq_k = gids == b_k
        gates.append(gate_k)
        onehots.append(eq_k.astype(jnp.float32))
        bmaxs.append(jnp.max(jnp.where(eq_k, gate_k, -1e30), axis=1,
                             keepdims=True))
    m_new = jnp.maximum(jnp.maximum(jnp.maximum(m_old, bmaxs[0]),
                                    jnp.maximum(bmaxs[1], bmaxs[2])),
                        bmaxs[3])
    scale = jnp.exp(m_old - m_new)
    gsum_inc = jnp.zeros((NG, 1), jnp.float32)
    pooled_inc = jnp.zeros((NG, GRPH), jnp.float32)
    for k in range(4):
        m_node = lax.dot_general(m_new, onehots[k], (((0,), (0,)), ((), ())),
                                 preferred_element_type=jnp.float32)
        wmat = jnp.where(onehots[k] > 0.0,
                         jnp.exp(gates[k] - m_node), 0.0)
        gsum_inc = gsum_inc + jnp.sum(wmat, axis=1, keepdims=True)
        pooled_inc = pooled_inc + mm(wmat, v[:, k * GRPH:(k + 1) * GRPH])
    m_ref[...] = m_new
    gsum_ref[...] = gsum_ref[...] * scale + gsum_inc
    pooled_ref[...] = pooled_ref[...] * scale + pooled_inc

    @pl.when(i == GB - 1)
    def _fin():
        pooled = pooled_ref[...] / (gsum_ref[...] + 1e-16)
        logits = mm(pooled, wo[...]) + bo[...]
        lane = lax.broadcasted_iota(jnp.int32, (NG, 128), 1)
        logits = jnp.where(lane < 2, logits, -1e30)
        mx = jnp.max(logits, axis=1, keepdims=True)
        p = jnp.exp(logits - mx)
        out_ref[...] = p / jnp.sum(p, axis=1, keepdims=True)


def _final(x0s, x1s, x2s, batch, We0, be0, We1, be1, We2, be2, Wm, bm,
           Wg, bg, Wv, bv, Wo, bo):
    half = pl.BlockSpec((1, PB, 128), lambda i: (0, i, 0))
    half2 = pl.BlockSpec((1, PB, 128), lambda i: (1, i, 0))
    wspec = pl.BlockSpec((128, 4 * GRPH), lambda i: (0, 0))
    bspec = pl.BlockSpec((1, 4 * GRPH), lambda i: (0, 0))
    mspec = pl.BlockSpec((4 * DH, 4 * GRPH), lambda i: (0, 0))
    Wo128 = jnp.pad(Wo, ((0, 0), (0, 128 - Wo.shape[1])))
    bo128 = jnp.pad(bo.reshape(1, -1), ((0, 0), (0, 128 - bo.shape[0])))
    batp = jnp.concatenate(
        [batch, jnp.full((4 * PK - N,), -1, jnp.int32)]
    ).reshape(GB, PB, 4).transpose(0, 2, 1)
    out = pl.pallas_call(
        _final_body,
        grid=(GB,),
        in_specs=[
            half, half2, half, half2, half, half2,
            pl.BlockSpec((1, 4, PB), lambda i: (i, 0, 0)),
            wspec, wspec, bspec, wspec, wspec, bspec, wspec, wspec, bspec,
            mspec, mspec, mspec, bspec,
            pl.BlockSpec((DH, 1), lambda i: (0, 0)),
            pl.BlockSpec((1, 1), lambda i: (0, 0)),
            mspec,
            bspec,
            pl.BlockSpec((DH, 128), lambda i: (0, 0)),
            pl.BlockSpec((1, 128), lambda i: (0, 0)),
        ],
        out_specs=pl.BlockSpec((NG, 128), lambda i: (0, 0)),
        out_shape=jax.ShapeDtypeStruct((NG, 128), jnp.float32),
        compiler_params=pltpu.CompilerParams(
            dimension_semantics=("arbitrary",)),
        scratch_shapes=[
            pltpu.VMEM((NG, 1), jnp.float32),
            pltpu.VMEM((NG, 1), jnp.float32),
            pltpu.VMEM((NG, GRPH), jnp.float32),
        ],
    )(x0s, x0s, x1s, x1s, x2s, x2s, batp,
      _bd4(We0[:HF]), _bd4(We0[HF:]), _t4(be0),
      _bd4(We1[:HF]), _bd4(We1[HF:]), _t4(be1),
      _bd4(We2[:HF]), _bd4(We2[HF:]), _t4(be2),
      _bd4(Wm[:DH]), _bd4(Wm[DH:2 * DH]), _bd4(Wm[2 * DH:]), _t4(bm),
      Wg, bg.reshape(1, 1), _bd4(Wv), _t4(bv), Wo128, bo128)
    return out[:, :2]


def _sage_agg(xs, ei, with_deg):
    if with_deg:
        return tuple(_make_sc_agg(True)(xs, ei))
    return tuple(_make_sc_agg(False)(xs, ei))


def kernel(x, edge_index, batch, W0, b0, W1l, b1l, W1r, W2l, b2l, W2r,
           We0, be0, We1, be1, We2, be2, Wm, bm, Wg, bg, Wv, bv, Wo, bo):
    pad_i = jnp.arange(EP - E, dtype=jnp.int32)
    pads = jnp.stack([pad_i % N, N + pad_i % (SACC - N)])
    ei = jnp.concatenate([edge_index, pads], axis=1).reshape(2, EROWS, 128)

    x0s = _dense0(x, W0, b0)
    agg1, degp = _sage_agg(x0s.reshape(2, ACC_ROWS, HF), ei, True)
    degq = jnp.pad(degp, (0, 4 * PK - SACC)).reshape(PK, 4)
    sel = jnp.kron(jnp.eye(4, dtype=jnp.float32), jnp.ones((1, HF), jnp.float32))
    x1s = _dense_sage(agg1, degq, sel, x0s, W1l, b1l, W1r)
    (agg2,) = _sage_agg(x1s.reshape(2, ACC_ROWS, HF), ei, False)
    x2s = _dense_sage(agg2, degq, sel, x1s, W2l, b2l, W2r)
    return _final(x0s, x1s, x2s, batch, We0, be0, We1, be1, We2, be2,
                  Wm, bm, Wg, bg, Wv, bv, Wo, bo)


# speedup vs baseline: 15.2645x; 1.0059x over previous
import functools

import jax
import jax.numpy as jnp
from jax import lax
from jax.experimental import pallas as pl
from jax.experimental.pallas import tpu as pltpu
from jax.experimental.pallas import tpu_sc as plsc

N = 50000
E = 800000
NG = 256
DIN = 128
DH = 64
HF = 32
GRPH = 64

EP = 819200
EROWS = EP // 128
NC = 2
NS = 16
RPT = EROWS // (NS)
K = 2
CHUNKS = RPT // K
TRIPS = 66
SACC = 50176
SAPT = SACC // NS
ACC_ROWS = 51200
ZROWS = 64
APT = ACC_ROWS // NS

RB = 2000
GB = N // RB
PB = 512
PK = ACC_ROWS // 4
PN = N // 4
PKV = SACC // 4


def _bd4(W):
    return jnp.kron(jnp.eye(4, dtype=W.dtype), W)


def _t4(b):
    return jnp.tile(b, 4).reshape(1, -1)


def _mm0_body(x_ref, wa_ref, wb_ref, ba_ref, bb_ref, o_ref):
    xb = x_ref[...]
    o_ref[0] = jnp.dot(xb, wa_ref[...],
                       preferred_element_type=jnp.float32) + ba_ref[...]
    o_ref[1] = jnp.dot(xb, wb_ref[...],
                       preferred_element_type=jnp.float32) + bb_ref[...]


def _dense0(x, W0, b0):
    xp = x.reshape(PN, 4 * DIN)
    wspec = pl.BlockSpec((4 * DIN, 128), lambda i: (0, 0))
    bspec = pl.BlockSpec((1, 128), lambda i: (0, 0))
    return pl.pallas_call(
        _mm0_body,
        grid=(GB,),
        in_specs=[
            pl.BlockSpec((PB, 4 * DIN), lambda i: (i, 0)),
            wspec, wspec, bspec, bspec,
        ],
        out_specs=pl.BlockSpec((2, PB, 128), lambda i: (0, i, 0)),
        out_shape=jax.ShapeDtypeStruct((2, PK, 128), jnp.float32),
    )(xp, _bd4(W0[:, :HF]), _bd4(W0[:, HF:]), _t4(b0[:HF]), _t4(b0[HF:]))


def _sage_body(aa_ref, ab_ref, dq_ref, sel_ref, xa_ref, xb_ref,
               wlaa, wlba, wraa, wrba, wlab, wlbb, wrab, wrbb,
               ba_ref, bb_ref, o_ref):
    def mm(a, b):
        return jnp.dot(a, b[...], preferred_element_type=jnp.float32)
    dpk = jnp.dot(dq_ref[...], sel_ref[...],
                  preferred_element_type=jnp.float32)
    inv = 1.0 / jnp.maximum(dpk, 1.0)
    na = aa_ref[0] * inv
    nb = ab_ref[0] * inv
    xa = xa_ref[0]
    xb = xb_ref[0]
    o_ref[0] = (mm(na, wlaa) + mm(nb, wlba) + mm(xa, wraa) + mm(xb, wrba)
                + ba_ref[...])
    o_ref[1] = (mm(na, wlab) + mm(nb, wlbb) + mm(xa, wrab) + mm(xb, wrbb)
                + bb_ref[...])


def _dense_sage(aggs, degq, sel, xs, Wl, bl, Wr):
    half = pl.BlockSpec((1, PB, 128), lambda i: (0, i, 0))
    half2 = pl.BlockSpec((1, PB, 128), lambda i: (1, i, 0))
    wspec = pl.BlockSpec((128, 128), lambda i: (0, 0))
    bspec = pl.BlockSpec((1, 128), lambda i: (0, 0))
    aggp = aggs.reshape(2, PK, 128)
    ws = [_bd4(Wl[:HF, :HF]), _bd4(Wl[HF:, :HF]),
          _bd4(Wr[:HF, :HF]), _bd4(Wr[HF:, :HF]),
          _bd4(Wl[:HF, HF:]), _bd4(Wl[HF:, HF:]),
          _bd4(Wr[:HF, HF:]), _bd4(Wr[HF:, HF:])]
    return pl.pallas_call(
        _sage_body,
        grid=(GB,),
        in_specs=[half, half2, pl.BlockSpec((PB, 4), lambda i: (i, 0)),
                  pl.BlockSpec((4, 128), lambda i: (0, 0)),
                  half, half2] + [wspec] * 8 + [bspec, bspec],
        out_specs=pl.BlockSpec((2, PB, 128), lambda i: (0, i, 0)),
        out_shape=jax.ShapeDtypeStruct((2, PK, 128), jnp.float32),
    )(aggp, aggp, degq, sel, xs, xs, *ws, _t4(bl[:HF]), _t4(bl[HF:]))


def _make_sc_agg(with_deg):
    mesh = plsc.VectorSubcoreMesh(core_axis_name="c", subcore_axis_name="s")
    out_type = [jax.ShapeDtypeStruct((2, ACC_ROWS, HF), jnp.float32)]
    scratch = [
        pltpu.VMEM((2, K * 128), jnp.int32),
        pltpu.VMEM((2, K * 128), jnp.int32),
        pltpu.VMEM((2, K * 128), jnp.int32),
        pltpu.VMEM((K * 128, HF), jnp.float32),
        pltpu.VMEM((K * 128, HF), jnp.float32),
        pltpu.VMEM((K * 128, HF), jnp.float32),
        pltpu.VMEM_SHARED((SACC, HF), jnp.float32),
        pltpu.SemaphoreType.DMA,
        pltpu.SemaphoreType.DMA,
        pltpu.SemaphoreType.DMA,
        pltpu.SemaphoreType.DMA,
        pltpu.SemaphoreType.DMA,
        pltpu.SemaphoreType.DMA,
        pltpu.SemaphoreType.DMA,
        pltpu.SemaphoreType.DMA,
        pltpu.SemaphoreType.DMA,
    ]
    if with_deg:
        out_type.append(jax.ShapeDtypeStruct((SACC,), jnp.float32))
        scratch += [
            pltpu.VMEM((448,), jnp.float32),
            pltpu.VMEM((K * 128,), jnp.float32),
            pltpu.VMEM_SHARED((SACC,), jnp.float32),
        ]

    def body(tbl_ref, ei_ref, out_ref, *rest):
        if with_deg:
            (deg_out, sd0, sd1, sd2, rows0, rows1, rows2, acc,
             isem0, isem1, isem2, gsem0, gsem1, gsem2, ssem0, ssem1, ssem2,
             zbuf1, ones, dacc) = rest
        else:
            (sd0, sd1, sd2, rows0, rows1, rows2, acc,
             isem0, isem1, isem2, gsem0, gsem1, gsem2, ssem0, ssem1, ssem2) = rest
        c = lax.axis_index("c")
        s = lax.axis_index("s")
        mytbl = tbl_ref.at[c]
        myout = out_ref.at[c]
        sds = [sd0, sd1, sd2]
        rows = [rows0, rows1, rows2]
        isems = [isem0, isem1, isem2]
        gsems = [gsem0, gsem1, gsem2]
        ssems = [ssem0, ssem1, ssem2]

        zero16 = jnp.zeros((16,), jnp.float32)

        NZ = K * 128
        @pl.loop(0, NZ)
        def _zfill(r):
            rows0[r, pl.ds(0, 16)] = zero16
            rows0[r, pl.ds(16, 16)] = zero16

        @pl.loop(0, SAPT // NZ)
        def _zacc(r):
            pltpu.sync_copy(rows0, acc.at[pl.ds(s * SAPT + r * NZ, NZ)])
        pltpu.sync_copy(rows0.at[pl.ds(0, SAPT % NZ)],
                        acc.at[pl.ds(s * SAPT + (SAPT // NZ) * NZ, SAPT % NZ)])

        if with_deg:
            @pl.when(c == 0)
            def _zdeg():
                @pl.loop(0, 28)
                def _zf1(r):
                    zbuf1[pl.ds(r * 16, 16)] = zero16
                one16 = jnp.ones((16,), jnp.float32)

                @pl.loop(0, K * 8)
                def _of(r):
                    ones[pl.ds(r * 16, 16)] = one16
                @pl.loop(0, 7)
                def _zd(r):
                    pltpu.sync_copy(zbuf1, dacc.at[pl.ds(s * SAPT + r * 448, 448)])

        plsc.subcore_barrier()

        def fire_idx(chunk_base, q):
            pltpu.async_copy(ei_ref.at[0, chunk_base], sds[q].at[0], isems[q])
            d = pltpu.async_copy(ei_ref.at[1, chunk_base], sds[q].at[1],
                                 isems[q])
            class _W:
                def wait(self):
                    pltpu.make_async_copy(ei_ref.at[0, chunk_base],
                                          sds[q].at[0], isems[q]).wait()
                    d.wait()
            return _W()

        def fire_gathers(q):
            pltpu.async_copy(mytbl.at[sds[q].at[0]], rows[q], gsems[q])

        def wait_gathers(q):
            pltpu.make_async_copy(mytbl.at[sds[q].at[0]], rows[q],
                                  gsems[q]).wait()

        def fire_scatters(q):
            pltpu.async_copy(rows[q], acc.at[sds[q].at[1]], ssems[q], add=True)
            if with_deg:
                @pl.when(c == 0)
                def _dfire():
                    pltpu.async_copy(ones, dacc.at[sds[q].at[1]],
                                     ssems[q], add=True)

        def wait_scatters(q):
            pltpu.make_async_copy(rows[q], acc.at[sds[q].at[1]],
                                  ssems[q]).wait()
            if with_deg:
                @pl.when(c == 0)
                def _dwait():
                    pltpu.make_async_copy(ones, dacc.at[sds[q].at[1]],
                                          ssems[q]).wait()

        tbase = s * (RPT // 2)
        for q in range(3):
            fire_idx(tbase + q, q).wait()
            fire_gathers(q)

        @pl.loop(0, TRIPS)
        def _trip(h):
            base = tbase + 3 * h
            for q in range(3):
                wait_gathers(q)
                fire_scatters(q)
            for q in range(3):
                wait_scatters(q)
                if q < 2:
                    fire_idx(base + q + 3, q).wait()
                    fire_gathers(q)
                else:
                    @pl.when(h < TRIPS - 1)
                    def _nextC():
                        fire_idx(base + 5, 2).wait()
                        fire_gathers(2)

        for q in range(2):
            wait_gathers(q)
            fire_scatters(q)
        for q in range(2):
            wait_scatters(q)

        plsc.subcore_barrier()

        pltpu.sync_copy(acc.at[pl.ds(s * SAPT, SAPT)],
                        myout.at[pl.ds(s * SAPT, SAPT)])
        if with_deg:
            @pl.when(c == 0)
            def _wdeg():
                pltpu.sync_copy(dacc.at[pl.ds(s * SAPT, SAPT)],
                                deg_out.at[pl.ds(s * SAPT, SAPT)])

    return pl.kernel(body, out_type=out_type, mesh=mesh, scratch_types=scratch,
                     compiler_params=pltpu.CompilerParams(
                         use_tc_tiling_on_sc=False))


_make_sc_agg = functools.cache(_make_sc_agg)


def _final_body(x0a, x0b, x1a, x1b, x2a, x2b, bat_ref,
                we0t, we0b, be0, we1t, we1b, be1, we2t, we2b, be2,
                wm0, wm1, wm2, bm, wg, bg, wv, bv, wo, bo,
                out_ref, m_ref, gsum_ref, pooled_ref):
    i = pl.program_id(0)

    @pl.when(i == 0)
    def _init():
        m_ref[...] = jnp.full((NG, 1), -1e30, jnp.float32)
        gsum_ref[...] = jnp.zeros((NG, 1), jnp.float32)
        pooled_ref[...] = jnp.zeros((NG, GRPH), jnp.float32)

    def mm(a, b):
        return jnp.dot(a, b, preferred_element_type=jnp.float32)

    e0 = jax.nn.relu(mm(x0a[0], we0t[...]) + mm(x0b[0], we0b[...]) + be0[...])
    e1 = jax.nn.relu(mm(x1a[0], we1t[...]) + mm(x1b[0], we1b[...]) + be1[...])
    e2 = jax.nn.relu(mm(x2a[0], we2t[...]) + mm(x2b[0], we2b[...]) + be2[...])
    xc = jax.nn.relu(mm(e0, wm0[...]) + mm(e1, wm1[...]) + mm(e2, wm2[...])
                     + bm[...])
    v = mm(xc, wv[...]) + bv[...]
    rid = i * PB + lax.broadcasted_iota(jnp.int32, (PB, 1), 0)
    v = jnp.where(rid < PN, v, 0.0)

    gids = lax.broadcasted_iota(jnp.int32, (NG, PB), 0)
    m_old = m_ref[...]
    gates, onehots, bmaxs = [], [], []
    for k in range(4):
        xck = xc[:, k * GRPH:(k + 1) * GRPH]
        gate_k = lax.dot_general(wg[...], xck, (((0,), (1,)), ((), ())),
                                 preferred_element_type=jnp.float32) + bg[...]
        b_k = bat_ref[0, pl.ds(k, 1)]
        eq_k = gids == b_k
        gates.append(gate_k)
        onehots.append(eq_k.astype(jnp.float32))
        bmaxs.append(jnp.max(jnp.where(eq_k, gate_k, -1e30), axis=1,
                             keepdims=True))
    m_new = jnp.maximum(jnp.maximum(jnp.maximum(m_old, bmaxs[0]),
                                    jnp.maximum(bmaxs[1], bmaxs[2])),
                        bmaxs[3])
    scale = jnp.exp(m_old - m_new)
    gsum_inc = jnp.zeros((NG, 1), jnp.float32)
    pooled_inc = jnp.zeros((NG, GRPH), jnp.float32)
    for k in range(4):
        m_node = lax.dot_general(m_new, onehots[k], (((0,), (0,)), ((), ())),
                                 preferred_element_type=jnp.float32)
        wmat = jnp.where(onehots[k] > 0.0,
                         jnp.exp(gates[k] - m_node), 0.0)
        gsum_inc = gsum_inc + jnp.sum(wmat, axis=1, keepdims=True)
        pooled_inc = pooled_inc + mm(wmat, v[:, k * GRPH:(k + 1) * GRPH])
    m_ref[...] = m_new
    gsum_ref[...] = gsum_ref[...] * scale + gsum_inc
    pooled_ref[...] = pooled_ref[...] * scale + pooled_inc

    @pl.when(i == GB - 1)
    def _fin():
        pooled = pooled_ref[...] / (gsum_ref[...] + 1e-16)
        logits = mm(pooled, wo[...]) + bo[...]
        lane = lax.broadcasted_iota(jnp.int32, (NG, 128), 1)
        logits = jnp.where(lane < 2, logits, -1e30)
        mx = jnp.max(logits, axis=1, keepdims=True)
        p = jnp.exp(logits - mx)
        out_ref[...] = p / jnp.sum(p, axis=1, keepdims=True)


def _final(x0s, x1s, x2s, batch, We0, be0, We1, be1, We2, be2, Wm, bm,
           Wg, bg, Wv, bv, Wo, bo):
    half = pl.BlockSpec((1, PB, 128), lambda i: (0, i, 0))
    half2 = pl.BlockSpec((1, PB, 128), lambda i: (1, i, 0))
    wspec = pl.BlockSpec((128, 4 * GRPH), lambda i: (0, 0))
    bspec = pl.BlockSpec((1, 4 * GRPH), lambda i: (0, 0))
    mspec = pl.BlockSpec((4 * DH, 4 * GRPH), lambda i: (0, 0))
    Wo128 = jnp.pad(Wo, ((0, 0), (0, 128 - Wo.shape[1])))
    bo128 = jnp.pad(bo.reshape(1, -1), ((0, 0), (0, 128 - bo.shape[0])))
    batp = jnp.concatenate(
        [batch, jnp.full((4 * PK - N,), -1, jnp.int32)]
    ).reshape(GB, PB, 4).transpose(0, 2, 1)
    out = pl.pallas_call(
        _final_body,
        grid=(GB,),
        in_specs=[
            half, half2, half, half2, half, half2,
            pl.BlockSpec((1, 4, PB), lambda i: (i, 0, 0)),
            wspec, wspec, bspec, wspec, wspec, bspec, wspec, wspec, bspec,
            mspec, mspec, mspec, bspec,
            pl.BlockSpec((DH, 1), lambda i: (0, 0)),
            pl.BlockSpec((1, 1), lambda i: (0, 0)),
            mspec,
            bspec,
            pl.BlockSpec((DH, 128), lambda i: (0, 0)),
            pl.BlockSpec((1, 128), lambda i: (0, 0)),
        ],
        out_specs=pl.BlockSpec((NG, 128), lambda i: (0, 0)),
        out_shape=jax.ShapeDtypeStruct((NG, 128), jnp.float32),
        compiler_params=pltpu.CompilerParams(
            dimension_semantics=("arbitrary",)),
        scratch_shapes=[
            pltpu.VMEM((NG, 1), jnp.float32),
            pltpu.VMEM((NG, 1), jnp.float32),
            pltpu.VMEM((NG, GRPH), jnp.float32),
        ],
    )(x0s, x0s, x1s, x1s, x2s, x2s, batp,
      _bd4(We0[:HF]), _bd4(We0[HF:]), _t4(be0),
      _bd4(We1[:HF]), _bd4(We1[HF:]), _t4(be1),
      _bd4(We2[:HF]), _bd4(We2[HF:]), _t4(be2),
      _bd4(Wm[:DH]), _bd4(Wm[DH:2 * DH]), _bd4(Wm[2 * DH:]), _t4(bm),
      Wg, bg.reshape(1, 1), _bd4(Wv), _t4(bv), Wo128, bo128)
    return out[:, :2]


def _sage_agg(xs, ei, with_deg):
    if with_deg:
        return tuple(_make_sc_agg(True)(xs, ei))
    return tuple(_make_sc_agg(False)(xs, ei))


def kernel(x, edge_index, batch, W0, b0, W1l, b1l, W1r, W2l, b2l, W2r,
           We0, be0, We1, be1, We2, be2, Wm, bm, Wg, bg, Wv, bv, Wo, bo):
    pad_i = jnp.arange(EP - E, dtype=jnp.int32)
    pads = jnp.stack([pad_i % N, N + pad_i % (SACC - N)])
    ei = jnp.concatenate([edge_index, pads], axis=1).reshape(2, EROWS // 2, 256)

    x0s = _dense0(x, W0, b0)
    agg1, degp = _sage_agg(x0s.reshape(2, ACC_ROWS, HF), ei, True)
    degq = jnp.pad(degp, (0, 4 * PK - SACC)).reshape(PK, 4)
    sel = jnp.kron(jnp.eye(4, dtype=jnp.float32), jnp.ones((1, HF), jnp.float32))
    x1s = _dense_sage(agg1, degq, sel, x0s, W1l, b1l, W1r)
    (agg2,) = _sage_agg(x1s.reshape(2, ACC_ROWS, HF), ei, False)
    x2s = _dense_sage(agg2, degq, sel, x1s, W2l, b2l, W2r)
    return _final(x0s, x1s, x2s, batch, We0, be0, We1, be1, We2, be2,
                  Wm, bm, Wg, bg, Wv, bv, Wo, bo)
